# Initial kernel scaffold; baseline (speedup 1.0000x reference)
import jax
import jax.numpy as jnp
from jax.experimental import pallas as pl


def kernel(h_src, h_dst, edge_index, edge_attr, n_dst, W_msg1, b_msg1, W_msg2, b_msg2, W_upd1, b_upd1, W_upd2, b_upd2, gamma, beta):
    raise NotImplementedError("write your pallas kernel here")


# speedup vs baseline: 1.9236x; 1.9236x over previous
import functools

import jax
import jax.numpy as jnp
from jax import lax
from jax.experimental import pallas as pl
from jax.experimental.pallas import tpu as pltpu
from jax.experimental.pallas import tpu_sc as plsc

HID = 128
NCORE = 2
NSUB = 16
NW = NCORE * NSUB
BE = 128
CH = 8


def _silu(x):
    return x / (1.0 + jnp.exp(-x))


def _pre_node_body(hs_ref, hd_ref, w1a_ref, w1b_ref, b1_ref, ps_ref, pd_ref):
    ps_ref[...] = jnp.dot(hs_ref[...], w1a_ref[...],
                          preferred_element_type=jnp.float32)
    pd_ref[...] = jnp.dot(hd_ref[...], w1b_ref[...],
                          preferred_element_type=jnp.float32) + b1_ref[...]


def _pre_edge_body(ea_ref, w1c_ref, e_ref):
    e_ref[...] = jnp.dot(ea_ref[...], w1c_ref[...],
                         preferred_element_type=jnp.float32)


def _zero_fill(a_v, ref, s, stripe):
    nrep = stripe // BE
    rem = stripe % BE
    for t in range(nrep):
        pltpu.sync_copy(a_v, ref.at[pl.ds(s * stripe + t * BE, BE)])
    if rem:
        pltpu.sync_copy(a_v.at[pl.ds(0, rem)],
                        ref.at[pl.ds(s * stripe + nrep * BE, rem)])


def _sc_edge_body(n_pad, nchunk, ps_hbm, pd_hbm, e_hbm, src_hbm, dst_hbm,
                  outs_hbm, g_hbm,
                  idx_v, a_v, b_v,
                  acc_s, sem_a, sem_b):
    c = lax.axis_index("c")
    s = lax.axis_index("s")
    wid = c * NSUB + s
    stripe = n_pad // NSUB
    epw = nchunk * CH * BE
    imax = jnp.int32(n_pad - 1)

    zeros = jnp.zeros((16,), jnp.float32)

    def _zrow(r, carry):
        for cc in range(8):
            a_v[r, pl.ds(cc * 16, 16)] = zeros
        return carry

    lax.fori_loop(0, BE, _zrow, 0)
    _zero_fill(a_v, acc_s, s, stripe)
    plsc.subcore_barrier()

    def _clamp(i, carry):
        r = i >> 3
        cc = (i & 7) * 16
        v = idx_v[r, pl.ds(cc, 16)]
        idx_v[r, pl.ds(cc, 16)] = jnp.minimum(jnp.maximum(v, 0), imax)
        return carry

    def _chunk1(t, carry):
        pltpu.sync_copy(src_hbm.at[wid, t], idx_v)
        lax.fori_loop(0, CH * 8, _clamp, 0)

        def _block(j, inner):
            rowb = wid * epw + (t * CH + j) * BE
            cp_a = pltpu.async_copy(ps_hbm.at[idx_v.at[j]], a_v, sem_a)
            cp_b = pltpu.async_copy(e_hbm.at[pl.ds(rowb, BE)], b_v, sem_b)
            cp_a.wait()
            cp_b.wait()

            def _row(r, inner2):
                for cc in range(8):
                    sl = pl.ds(cc * 16, 16)
                    a_v[r, sl] = a_v[r, sl] + b_v[r, sl]
                return inner2

            lax.fori_loop(0, BE, _row, 0)
            pltpu.sync_copy(a_v, g_hbm.at[pl.ds(rowb, BE)])
            return inner

        lax.fori_loop(0, CH, _block, 0)
        return carry

    lax.fori_loop(0, nchunk, _chunk1, 0)

    def _chunk2(t, carry):
        pltpu.sync_copy(dst_hbm.at[wid, t], idx_v)
        lax.fori_loop(0, CH * 8, _clamp, 0)

        def _block(j, inner):
            rowb = wid * epw + (t * CH + j) * BE
            cp_a = pltpu.async_copy(g_hbm.at[pl.ds(rowb, BE)], a_v, sem_a)
            cp_b = pltpu.async_copy(pd_hbm.at[idx_v.at[j]], b_v, sem_b)
            cp_a.wait()
            cp_b.wait()

            def _row(r, inner2):
                for cc in range(8):
                    sl = pl.ds(cc * 16, 16)
                    tt = a_v[r, sl] + b_v[r, sl]
                    a_v[r, sl] = tt / (1.0 + jnp.exp(-tt))
                return inner2

            lax.fori_loop(0, BE, _row, 0)
            pltpu.sync_copy(a_v, acc_s.at[idx_v.at[j]], add=True)
            return inner

        lax.fori_loop(0, CH, _block, 0)
        return carry

    lax.fori_loop(0, nchunk, _chunk2, 0)
    plsc.subcore_barrier()

    base = s * stripe
    pltpu.sync_copy(acc_s.at[pl.ds(base, stripe)],
                    outs_hbm.at[c, pl.ds(base, stripe)])


def _sc_count_body(n_pad, nchunk, dst_hbm, outc_hbm,
                   idx_v, one_v, acc_c):
    c = lax.axis_index("c")
    s = lax.axis_index("s")
    wid = c * NSUB + s
    stripe = n_pad // NSUB
    imax = jnp.int32(n_pad - 1)

    zeros = jnp.zeros((16,), jnp.float32)

    def _zrow(r, carry):
        for cc in range(8):
            one_v[r, pl.ds(cc * 16, 16)] = zeros
        return carry

    lax.fori_loop(0, BE, _zrow, 0)
    _zero_fill(one_v, acc_c, s, stripe)

    ones = jnp.ones((16,), jnp.float32)

    def _orow(r, carry):
        one_v[r, pl.ds(0, 16)] = ones
        return carry

    lax.fori_loop(0, BE, _orow, 0)
    plsc.subcore_barrier()

    def _clamp(i, carry):
        r = i >> 3
        cc = (i & 7) * 16
        v = idx_v[r, pl.ds(cc, 16)]
        idx_v[r, pl.ds(cc, 16)] = jnp.minimum(jnp.maximum(v, 0), imax)
        return carry

    def _chunk(t, carry):
        pltpu.sync_copy(dst_hbm.at[wid, t], idx_v)
        lax.fori_loop(0, CH * 8, _clamp, 0)

        def _block(j, inner):
            pltpu.sync_copy(one_v, acc_c.at[idx_v.at[j]], add=True)
            return inner

        lax.fori_loop(0, CH, _block, 0)
        return carry

    lax.fori_loop(0, nchunk, _chunk, 0)
    plsc.subcore_barrier()

    base = s * stripe
    pltpu.sync_copy(acc_c.at[pl.ds(base, stripe)],
                    outc_hbm.at[c, pl.ds(base, stripe)])


def _post_body(accs_ref, accc_ref, hd_ref, w2t_ref, b2_ref,
               wu1a_ref, wu1b_ref, bu1_ref, wu2t_ref, bu2_ref,
               g_ref, bt_ref, out_ref):
    seg = accs_ref[0] + accs_ref[1]
    cnt = (accc_ref[0] + accc_ref[1])[:, :1]
    agg_sum = jnp.dot(seg, w2t_ref[...],
                      preferred_element_type=jnp.float32) + cnt * b2_ref[...]
    agg = agg_sum / jnp.maximum(cnt, 1.0)
    hd = hd_ref[...]
    pre = (jnp.dot(hd, wu1a_ref[...], preferred_element_type=jnp.float32)
           + jnp.dot(agg, wu1b_ref[...], preferred_element_type=jnp.float32)
           + bu1_ref[...])
    dmid = _silu(pre)
    delta = jnp.dot(dmid, wu2t_ref[...],
                    preferred_element_type=jnp.float32) + bu2_ref[...]
    y = hd + delta
    mean = jnp.mean(y, axis=1, keepdims=True)
    d = y - mean
    var = jnp.mean(d * d, axis=1, keepdims=True)
    out_ref[...] = d * lax.rsqrt(var + 1e-5) * g_ref[...] + bt_ref[...]


def kernel(h_src, h_dst, edge_index, edge_attr, n_dst,
           W_msg1, b_msg1, W_msg2, b_msg2,
           W_upd1, b_upd1, W_upd2, b_upd2, gamma, beta):
    n_nodes = h_dst.shape[0]
    n_edges = edge_attr.shape[0]
    ef = edge_attr.shape[1]
    epw = n_edges // NW
    assert epw * NW == n_edges

    cbe = CH * BE
    epw_pad = -(-epw // cbe) * cbe
    nchunk = epw_pad // cbe
    stripe = -(-(-(-n_nodes // NSUB)) // 8) * 8
    n_pad = stripe * NSUB

    w1t = W_msg1.T
    w1a = w1t[:HID]
    w1b = w1t[HID:2 * HID]
    w1c = w1t[2 * HID:]

    f32 = jnp.float32
    hs_p = jnp.pad(h_src, ((0, n_pad - n_nodes), (0, 0)))
    hd_p = jnp.pad(h_dst, ((0, n_pad - n_nodes), (0, 0)))

    row_n = n_pad // 8
    p_src, p_dst = pl.pallas_call(
        _pre_node_body,
        grid=(8,),
        in_specs=[
            pl.BlockSpec((row_n, HID), lambda i: (i, 0)),
            pl.BlockSpec((row_n, HID), lambda i: (i, 0)),
            pl.BlockSpec((HID, HID), lambda i: (0, 0)),
            pl.BlockSpec((HID, HID), lambda i: (0, 0)),
            pl.BlockSpec((1, HID), lambda i: (0, 0)),
        ],
        out_specs=[
            pl.BlockSpec((row_n, HID), lambda i: (i, 0)),
            pl.BlockSpec((row_n, HID), lambda i: (i, 0)),
        ],
        out_shape=[
            jax.ShapeDtypeStruct((n_pad, HID), f32),
            jax.ShapeDtypeStruct((n_pad, HID), f32),
        ],
    )(hs_p, hd_p, w1a, w1b, b_msg1.reshape(1, HID))

    ne_pad = NW * epw_pad
    ea_p = jnp.pad(edge_attr.reshape(NW, epw, ef),
                   ((0, 0), (0, epw_pad - epw), (0, 0))).reshape(ne_pad, ef)
    idx_pad = jnp.int32(n_pad - 1)
    src_r = jnp.pad(edge_index[0].astype(jnp.int32).reshape(NW, epw),
                    ((0, 0), (0, epw_pad - epw)),
                    constant_values=idx_pad).reshape(NW, nchunk, CH, BE)
    dst_r = jnp.pad(edge_index[1].astype(jnp.int32).reshape(NW, epw),
                    ((0, 0), (0, epw_pad - epw)),
                    constant_values=idx_pad).reshape(NW, nchunk, CH, BE)

    row_e = 4096
    grid_e = ne_pad // row_e
    assert grid_e * row_e == ne_pad
    e_proj = pl.pallas_call(
        _pre_edge_body,
        grid=(grid_e,),
        in_specs=[
            pl.BlockSpec((row_e, ef), lambda i: (i, 0)),
            pl.BlockSpec((ef, HID), lambda i: (0, 0)),
        ],
        out_specs=pl.BlockSpec((row_e, HID), lambda i: (i, 0)),
        out_shape=jax.ShapeDtypeStruct((ne_pad, HID), f32),
    )(ea_p, w1c)

    sc_edge = functools.partial(
        pl.kernel,
        out_type=[
            jax.ShapeDtypeStruct((NCORE, n_pad, HID), f32),
            jax.ShapeDtypeStruct((ne_pad, HID), f32),
        ],
        mesh=plsc.VectorSubcoreMesh(core_axis_name="c", subcore_axis_name="s"),
        scratch_types=[
            pltpu.MemorySpace.VMEM((CH, BE), jnp.int32),
            pltpu.MemorySpace.VMEM((BE, HID), f32),
            pltpu.MemorySpace.VMEM((BE, HID), f32),
            pltpu.MemorySpace.VMEM_SHARED((n_pad, HID), f32),
            pltpu.SemaphoreType.DMA,
            pltpu.SemaphoreType.DMA,
        ],
    )(functools.partial(_sc_edge_body, n_pad, nchunk))

    acc_s, _ = sc_edge(p_src, p_dst, e_proj, src_r, dst_r)

    sc_count = functools.partial(
        pl.kernel,
        out_type=jax.ShapeDtypeStruct((NCORE, n_pad, HID), f32),
        mesh=plsc.VectorSubcoreMesh(core_axis_name="c", subcore_axis_name="s"),
        scratch_types=[
            pltpu.MemorySpace.VMEM((CH, BE), jnp.int32),
            pltpu.MemorySpace.VMEM((BE, HID), f32),
            pltpu.MemorySpace.VMEM_SHARED((n_pad, HID), f32),
        ],
    )(functools.partial(_sc_count_body, n_pad, nchunk))

    acc_c = sc_count(dst_r)

    wu1t = W_upd1.T
    row_p = n_nodes // 10
    out = pl.pallas_call(
        _post_body,
        grid=(10,),
        in_specs=[
            pl.BlockSpec((NCORE, row_p, HID), lambda i: (0, i, 0)),
            pl.BlockSpec((NCORE, row_p, HID), lambda i: (0, i, 0)),
            pl.BlockSpec((row_p, HID), lambda i: (i, 0)),
            pl.BlockSpec((HID, HID), lambda i: (0, 0)),
            pl.BlockSpec((1, HID), lambda i: (0, 0)),
            pl.BlockSpec((HID, HID), lambda i: (0, 0)),
            pl.BlockSpec((HID, HID), lambda i: (0, 0)),
            pl.BlockSpec((1, HID), lambda i: (0, 0)),
            pl.BlockSpec((HID, HID), lambda i: (0, 0)),
            pl.BlockSpec((1, HID), lambda i: (0, 0)),
            pl.BlockSpec((1, HID), lambda i: (0, 0)),
            pl.BlockSpec((1, HID), lambda i: (0, 0)),
        ],
        out_specs=pl.BlockSpec((row_p, HID), lambda i: (i, 0)),
        out_shape=jax.ShapeDtypeStruct((n_nodes, HID), f32),
    )(acc_s, acc_c, h_dst, W_msg2.T, b_msg2.reshape(1, HID),
      wu1t[:HID], wu1t[HID:], b_upd1.reshape(1, HID),
      W_upd2.T, b_upd2.reshape(1, HID),
      gamma.reshape(1, HID), beta.reshape(1, HID))
    return out


# speedup vs baseline: 2.4306x; 1.2636x over previous
import functools

import jax
import jax.numpy as jnp
from jax import lax
from jax.experimental import pallas as pl
from jax.experimental.pallas import tpu as pltpu
from jax.experimental.pallas import tpu_sc as plsc

HID = 128
NCORE = 2
NSUB = 16
NW = NCORE * NSUB
BE = 128
CH = 8


def _silu(x):
    return x / (1.0 + jnp.exp(-x))


def _pre_node_body(hs_ref, hd_ref, w1a_ref, w1b_ref, b1_ref, ps_ref, pd_ref):
    ps_ref[...] = jnp.dot(hs_ref[...], w1a_ref[...],
                          preferred_element_type=jnp.float32)
    pd_ref[...] = jnp.dot(hd_ref[...], w1b_ref[...],
                          preferred_element_type=jnp.float32) + b1_ref[...]


def _pre_edge_body(ea_ref, w1c_ref, e_ref):
    e_ref[...] = jnp.dot(ea_ref[...], w1c_ref[...],
                         preferred_element_type=jnp.float32)


def _zero_fill(a_v, ref, s, stripe):
    nrep = stripe // BE
    rem = stripe % BE
    for t in range(nrep):
        pltpu.sync_copy(a_v, ref.at[pl.ds(s * stripe + t * BE, BE)])
    if rem:
        pltpu.sync_copy(a_v.at[pl.ds(0, rem)],
                        ref.at[pl.ds(s * stripe + nrep * BE, rem)])


def _sc_edge_body(n_pad, nchunk, ps_hbm, pd_hbm, e_hbm, src_hbm, dst_hbm,
                  outs_hbm,
                  si_v, di_v, a_v, b_v,
                  acc_s, sem_a, sem_b):
    c = lax.axis_index("c")
    s = lax.axis_index("s")
    wid = c * NSUB + s
    stripe = n_pad // NSUB
    epw = nchunk * CH * BE
    imax = jnp.int32(n_pad - 1)

    zeros = jnp.zeros((16,), jnp.float32)

    def _zrow(r, carry):
        for cc in range(8):
            a_v[r, pl.ds(cc * 16, 16)] = zeros
        return carry

    lax.fori_loop(0, BE, _zrow, 0)
    _zero_fill(a_v, acc_s, s, stripe)
    plsc.subcore_barrier()

    def _mkclamp(idx_v):
        def _clamp(i, carry):
            r = i >> 3
            cc = (i & 7) * 16
            v = idx_v[r, pl.ds(cc, 16)]
            idx_v[r, pl.ds(cc, 16)] = jnp.minimum(jnp.maximum(v, 0), imax)
            return carry
        return _clamp

    def _chunk(t, carry):
        pltpu.sync_copy(src_hbm.at[wid, t], si_v)
        pltpu.sync_copy(dst_hbm.at[wid, t], di_v)
        lax.fori_loop(0, CH * 8, _mkclamp(si_v), 0)
        lax.fori_loop(0, CH * 8, _mkclamp(di_v), 0)

        def _block(j, inner):
            rowb = wid * epw + (t * CH + j) * BE
            cp_a = pltpu.async_copy(ps_hbm.at[si_v.at[j]], a_v, sem_a)
            cp_b = pltpu.async_copy(e_hbm.at[pl.ds(rowb, BE)], b_v, sem_b)
            cp_a.wait()
            cp_b.wait()

            def _row1(i, inner2):
                r = i * 2
                for rr in range(2):
                    for cc in range(8):
                        sl = pl.ds(cc * 16, 16)
                        a_v[r + rr, sl] = a_v[r + rr, sl] + b_v[r + rr, sl]
                return inner2

            lax.fori_loop(0, BE // 2, _row1, 0)

            cp_c = pltpu.async_copy(pd_hbm.at[di_v.at[j]], b_v, sem_b)
            cp_c.wait()

            def _row2(i, inner2):
                r = i * 2
                for rr in range(2):
                    for cc in range(8):
                        sl = pl.ds(cc * 16, 16)
                        tt = a_v[r + rr, sl] + b_v[r + rr, sl]
                        a_v[r + rr, sl] = tt / (1.0 + jnp.exp(-tt))
                return inner2

            lax.fori_loop(0, BE // 2, _row2, 0)
            pltpu.sync_copy(a_v, acc_s.at[di_v.at[j]], add=True)
            return inner

        lax.fori_loop(0, CH, _block, 0)
        return carry

    lax.fori_loop(0, nchunk, _chunk, 0)
    plsc.subcore_barrier()

    base = s * stripe
    pltpu.sync_copy(acc_s.at[pl.ds(base, stripe)],
                    outs_hbm.at[c, pl.ds(base, stripe)])


def _sc_count_body(n_pad, nchunk, dst_hbm, outc_hbm,
                   idx_v, one_v, acc_c):
    c = lax.axis_index("c")
    s = lax.axis_index("s")
    wid = c * NSUB + s
    stripe = n_pad // NSUB
    imax = jnp.int32(n_pad - 1)

    zeros = jnp.zeros((16,), jnp.float32)

    def _zrow(r, carry):
        for cc in range(8):
            one_v[r, pl.ds(cc * 16, 16)] = zeros
        return carry

    lax.fori_loop(0, BE, _zrow, 0)
    _zero_fill(one_v, acc_c, s, stripe)

    ones = jnp.ones((16,), jnp.float32)

    def _orow(r, carry):
        one_v[r, pl.ds(0, 16)] = ones
        return carry

    lax.fori_loop(0, BE, _orow, 0)
    plsc.subcore_barrier()

    def _clamp(i, carry):
        r = i >> 3
        cc = (i & 7) * 16
        v = idx_v[r, pl.ds(cc, 16)]
        idx_v[r, pl.ds(cc, 16)] = jnp.minimum(jnp.maximum(v, 0), imax)
        return carry

    def _chunk(t, carry):
        pltpu.sync_copy(dst_hbm.at[wid, t], idx_v)
        lax.fori_loop(0, CH * 8, _clamp, 0)

        def _block(j, inner):
            pltpu.sync_copy(one_v, acc_c.at[idx_v.at[j]], add=True)
            return inner

        lax.fori_loop(0, CH, _block, 0)
        return carry

    lax.fori_loop(0, nchunk, _chunk, 0)
    plsc.subcore_barrier()

    base = s * stripe
    pltpu.sync_copy(acc_c.at[pl.ds(base, stripe)],
                    outc_hbm.at[c, pl.ds(base, stripe)])


def _post_body(accs_ref, accc_ref, hd_ref, w2t_ref, b2_ref,
               wu1a_ref, wu1b_ref, bu1_ref, wu2t_ref, bu2_ref,
               g_ref, bt_ref, out_ref):
    seg = accs_ref[0] + accs_ref[1]
    cnt = (accc_ref[0] + accc_ref[1])[:, :1]
    agg_sum = jnp.dot(seg, w2t_ref[...],
                      preferred_element_type=jnp.float32) + cnt * b2_ref[...]
    agg = agg_sum / jnp.maximum(cnt, 1.0)
    hd = hd_ref[...]
    pre = (jnp.dot(hd, wu1a_ref[...], preferred_element_type=jnp.float32)
           + jnp.dot(agg, wu1b_ref[...], preferred_element_type=jnp.float32)
           + bu1_ref[...])
    dmid = _silu(pre)
    delta = jnp.dot(dmid, wu2t_ref[...],
                    preferred_element_type=jnp.float32) + bu2_ref[...]
    y = hd + delta
    mean = jnp.mean(y, axis=1, keepdims=True)
    d = y - mean
    var = jnp.mean(d * d, axis=1, keepdims=True)
    out_ref[...] = d * lax.rsqrt(var + 1e-5) * g_ref[...] + bt_ref[...]


def kernel(h_src, h_dst, edge_index, edge_attr, n_dst,
           W_msg1, b_msg1, W_msg2, b_msg2,
           W_upd1, b_upd1, W_upd2, b_upd2, gamma, beta):
    n_nodes = h_dst.shape[0]
    n_edges = edge_attr.shape[0]
    ef = edge_attr.shape[1]
    epw = n_edges // NW
    assert epw * NW == n_edges

    cbe = CH * BE
    epw_pad = -(-epw // cbe) * cbe
    nchunk = epw_pad // cbe
    stripe = -(-(-(-n_nodes // NSUB)) // 8) * 8
    n_pad = stripe * NSUB

    w1t = W_msg1.T
    w1a = w1t[:HID]
    w1b = w1t[HID:2 * HID]
    w1c = w1t[2 * HID:]

    f32 = jnp.float32
    hs_p = jnp.pad(h_src, ((0, n_pad - n_nodes), (0, 0)))
    hd_p = jnp.pad(h_dst, ((0, n_pad - n_nodes), (0, 0)))

    row_n = n_pad // 8
    p_src, p_dst = pl.pallas_call(
        _pre_node_body,
        grid=(8,),
        in_specs=[
            pl.BlockSpec((row_n, HID), lambda i: (i, 0)),
            pl.BlockSpec((row_n, HID), lambda i: (i, 0)),
            pl.BlockSpec((HID, HID), lambda i: (0, 0)),
            pl.BlockSpec((HID, HID), lambda i: (0, 0)),
            pl.BlockSpec((1, HID), lambda i: (0, 0)),
        ],
        out_specs=[
            pl.BlockSpec((row_n, HID), lambda i: (i, 0)),
            pl.BlockSpec((row_n, HID), lambda i: (i, 0)),
        ],
        out_shape=[
            jax.ShapeDtypeStruct((n_pad, HID), f32),
            jax.ShapeDtypeStruct((n_pad, HID), f32),
        ],
    )(hs_p, hd_p, w1a, w1b, b_msg1.reshape(1, HID))

    ne_pad = NW * epw_pad
    ea_p = jnp.pad(edge_attr.reshape(NW, epw, ef),
                   ((0, 0), (0, epw_pad - epw), (0, 0))).reshape(ne_pad, ef)
    idx_pad = jnp.int32(n_pad - 1)
    src_r = jnp.pad(edge_index[0].astype(jnp.int32).reshape(NW, epw),
                    ((0, 0), (0, epw_pad - epw)),
                    constant_values=idx_pad).reshape(NW, nchunk, CH, BE)
    dst_r = jnp.pad(edge_index[1].astype(jnp.int32).reshape(NW, epw),
                    ((0, 0), (0, epw_pad - epw)),
                    constant_values=idx_pad).reshape(NW, nchunk, CH, BE)

    row_e = 4096
    grid_e = ne_pad // row_e
    assert grid_e * row_e == ne_pad
    e_proj = pl.pallas_call(
        _pre_edge_body,
        grid=(grid_e,),
        in_specs=[
            pl.BlockSpec((row_e, ef), lambda i: (i, 0)),
            pl.BlockSpec((ef, HID), lambda i: (0, 0)),
        ],
        out_specs=pl.BlockSpec((row_e, HID), lambda i: (i, 0)),
        out_shape=jax.ShapeDtypeStruct((ne_pad, HID), f32),
    )(ea_p, w1c)

    sc_edge = functools.partial(
        pl.kernel,
        out_type=jax.ShapeDtypeStruct((NCORE, n_pad, HID), f32),
        mesh=plsc.VectorSubcoreMesh(core_axis_name="c", subcore_axis_name="s"),
        scratch_types=[
            pltpu.MemorySpace.VMEM((CH, BE), jnp.int32),
            pltpu.MemorySpace.VMEM((CH, BE), jnp.int32),
            pltpu.MemorySpace.VMEM((BE, HID), f32),
            pltpu.MemorySpace.VMEM((BE, HID), f32),
            pltpu.MemorySpace.VMEM_SHARED((n_pad, HID), f32),
            pltpu.SemaphoreType.DMA,
            pltpu.SemaphoreType.DMA,
        ],
    )(functools.partial(_sc_edge_body, n_pad, nchunk))

    acc_s = sc_edge(p_src, p_dst, e_proj, src_r, dst_r)

    sc_count = functools.partial(
        pl.kernel,
        out_type=jax.ShapeDtypeStruct((NCORE, n_pad, HID), f32),
        mesh=plsc.VectorSubcoreMesh(core_axis_name="c", subcore_axis_name="s"),
        scratch_types=[
            pltpu.MemorySpace.VMEM((CH, BE), jnp.int32),
            pltpu.MemorySpace.VMEM((BE, HID), f32),
            pltpu.MemorySpace.VMEM_SHARED((n_pad, HID), f32),
        ],
    )(functools.partial(_sc_count_body, n_pad, nchunk))

    acc_c = sc_count(dst_r)

    wu1t = W_upd1.T
    row_p = n_nodes // 10
    out = pl.pallas_call(
        _post_body,
        grid=(10,),
        in_specs=[
            pl.BlockSpec((NCORE, row_p, HID), lambda i: (0, i, 0)),
            pl.BlockSpec((NCORE, row_p, HID), lambda i: (0, i, 0)),
            pl.BlockSpec((row_p, HID), lambda i: (i, 0)),
            pl.BlockSpec((HID, HID), lambda i: (0, 0)),
            pl.BlockSpec((1, HID), lambda i: (0, 0)),
            pl.BlockSpec((HID, HID), lambda i: (0, 0)),
            pl.BlockSpec((HID, HID), lambda i: (0, 0)),
            pl.BlockSpec((1, HID), lambda i: (0, 0)),
            pl.BlockSpec((HID, HID), lambda i: (0, 0)),
            pl.BlockSpec((1, HID), lambda i: (0, 0)),
            pl.BlockSpec((1, HID), lambda i: (0, 0)),
            pl.BlockSpec((1, HID), lambda i: (0, 0)),
        ],
        out_specs=pl.BlockSpec((row_p, HID), lambda i: (i, 0)),
        out_shape=jax.ShapeDtypeStruct((n_nodes, HID), f32),
    )(acc_s, acc_c, h_dst, W_msg2.T, b_msg2.reshape(1, HID),
      wu1t[:HID], wu1t[HID:], b_upd1.reshape(1, HID),
      W_upd2.T, b_upd2.reshape(1, HID),
      gamma.reshape(1, HID), beta.reshape(1, HID))
    return out


# speedup vs baseline: 2.5721x; 1.0582x over previous
import functools

import jax
import jax.numpy as jnp
from jax import lax
from jax.experimental import pallas as pl
from jax.experimental.pallas import tpu as pltpu
from jax.experimental.pallas import tpu_sc as plsc

HID = 128
NCORE = 2
NSUB = 16
NW = NCORE * NSUB
BE = 128
CH = 8


def _silu(x):
    return x / (1.0 + jnp.exp(-x))


def _pre_node_body(hs_ref, hd_ref, w1a_ref, w1b_ref, b1_ref, ps_ref, pd_ref):
    ps_ref[...] = jnp.dot(hs_ref[...], w1a_ref[...],
                          preferred_element_type=jnp.float32)
    pd_ref[...] = jnp.dot(hd_ref[...], w1b_ref[...],
                          preferred_element_type=jnp.float32) + b1_ref[...]


def _pre_edge_body(ea_ref, w1c_ref, e_ref):
    e_ref[...] = jnp.dot(ea_ref[...], w1c_ref[...],
                         preferred_element_type=jnp.float32)


def _zero_fill(a_v, ref, s, stripe):
    nrep = stripe // BE
    rem = stripe % BE
    for t in range(nrep):
        pltpu.sync_copy(a_v, ref.at[pl.ds(s * stripe + t * BE, BE)])
    if rem:
        pltpu.sync_copy(a_v.at[pl.ds(0, rem)],
                        ref.at[pl.ds(s * stripe + nrep * BE, rem)])


def _sc_edge_body(n_pad, nchunk, ps_hbm, pd_hbm, e_hbm, src_hbm, dst_hbm,
                  outs_hbm,
                  si_v, di_v, a_v, b_v,
                  acc_s, sem_a, sem_b, sem_c):
    c = lax.axis_index("c")
    s = lax.axis_index("s")
    wid = c * NSUB + s
    stripe = n_pad // NSUB
    epw = nchunk * CH * BE
    imax = jnp.int32(n_pad - 1)

    zeros = jnp.zeros((16,), jnp.float32)

    def _zrow(r, carry):
        for cc in range(8):
            a_v[r, pl.ds(cc * 16, 16)] = zeros
        return carry

    lax.fori_loop(0, BE, _zrow, 0)
    _zero_fill(a_v, acc_s, s, stripe)
    plsc.subcore_barrier()

    def _mkclamp(idx_v):
        def _clamp(i, carry):
            r = i >> 3
            cc = (i & 7) * 16
            v = idx_v[r, pl.ds(cc, 16)]
            idx_v[r, pl.ds(cc, 16)] = jnp.minimum(jnp.maximum(v, 0), imax)
            return carry
        return _clamp

    def _chunk(t, carry):
        pltpu.sync_copy(src_hbm.at[wid, t], si_v)
        pltpu.sync_copy(dst_hbm.at[wid, t], di_v)
        lax.fori_loop(0, CH * 8, _mkclamp(si_v), 0)
        lax.fori_loop(0, CH * 8, _mkclamp(di_v), 0)

        def _addpass(base):
            def f(i, inner2):
                r = base + i * 2
                for rr in range(2):
                    for cc in range(8):
                        sl = pl.ds(cc * 16, 16)
                        a_v[r + rr, sl] = a_v[r + rr, sl] + b_v[r + rr, sl]
                return inner2
            lax.fori_loop(0, BE // 4, f, 0)

        def _silupass(base):
            def f(i, inner2):
                r = base + i * 2
                for rr in range(2):
                    for cc in range(8):
                        sl = pl.ds(cc * 16, 16)
                        tt = a_v[r + rr, sl] + b_v[r + rr, sl]
                        a_v[r + rr, sl] = tt / (1.0 + jnp.exp(-tt))
                return inner2
            lax.fori_loop(0, BE // 4, f, 0)

        H = BE // 2

        def _block(j, inner):
            rowb = wid * epw + (t * CH + j) * BE
            cps0 = pltpu.async_copy(ps_hbm.at[si_v.at[j, pl.ds(0, H)]],
                                    a_v.at[pl.ds(0, H)], sem_a)
            cpe0 = pltpu.async_copy(e_hbm.at[pl.ds(rowb, H)],
                                    b_v.at[pl.ds(0, H)], sem_b)
            cps1 = pltpu.async_copy(ps_hbm.at[si_v.at[j, pl.ds(H, H)]],
                                    a_v.at[pl.ds(H, H)], sem_a)
            cpe1 = pltpu.async_copy(e_hbm.at[pl.ds(rowb + H, H)],
                                    b_v.at[pl.ds(H, H)], sem_b)
            cps0.wait()
            cpe0.wait()
            _addpass(0)
            cpd0 = pltpu.async_copy(pd_hbm.at[di_v.at[j, pl.ds(0, H)]],
                                    b_v.at[pl.ds(0, H)], sem_c)
            cps1.wait()
            cpe1.wait()
            _addpass(H)
            cpd1 = pltpu.async_copy(pd_hbm.at[di_v.at[j, pl.ds(H, H)]],
                                    b_v.at[pl.ds(H, H)], sem_c)
            cpd0.wait()
            _silupass(0)
            cpd1.wait()
            _silupass(H)
            pltpu.sync_copy(a_v, acc_s.at[di_v.at[j]], add=True)
            return inner

        lax.fori_loop(0, CH, _block, 0)
        return carry

    lax.fori_loop(0, nchunk, _chunk, 0)
    plsc.subcore_barrier()

    base = s * stripe
    pltpu.sync_copy(acc_s.at[pl.ds(base, stripe)],
                    outs_hbm.at[c, pl.ds(base, stripe)])


def _sc_count_body(n_pad, nchunk, dst_hbm, outc_hbm,
                   idx_v, one_v, acc_c):
    c = lax.axis_index("c")
    s = lax.axis_index("s")
    wid = c * NSUB + s
    stripe = n_pad // NSUB
    imax = jnp.int32(n_pad - 1)

    zeros = jnp.zeros((16,), jnp.float32)

    def _zrow(r, carry):
        for cc in range(8):
            one_v[r, pl.ds(cc * 16, 16)] = zeros
        return carry

    lax.fori_loop(0, BE, _zrow, 0)
    _zero_fill(one_v, acc_c, s, stripe)

    ones = jnp.ones((16,), jnp.float32)

    def _orow(r, carry):
        one_v[r, pl.ds(0, 16)] = ones
        return carry

    lax.fori_loop(0, BE, _orow, 0)
    plsc.subcore_barrier()

    def _clamp(i, carry):
        r = i >> 3
        cc = (i & 7) * 16
        v = idx_v[r, pl.ds(cc, 16)]
        idx_v[r, pl.ds(cc, 16)] = jnp.minimum(jnp.maximum(v, 0), imax)
        return carry

    def _chunk(t, carry):
        pltpu.sync_copy(dst_hbm.at[wid, t], idx_v)
        lax.fori_loop(0, CH * 8, _clamp, 0)

        def _block(j, inner):
            pltpu.sync_copy(one_v, acc_c.at[idx_v.at[j]], add=True)
            return inner

        lax.fori_loop(0, CH, _block, 0)
        return carry

    lax.fori_loop(0, nchunk, _chunk, 0)
    plsc.subcore_barrier()

    base = s * stripe
    pltpu.sync_copy(acc_c.at[pl.ds(base, stripe)],
                    outc_hbm.at[c, pl.ds(base, stripe)])


def _post_body(accs_ref, accc_ref, hd_ref, w2t_ref, b2_ref,
               wu1a_ref, wu1b_ref, bu1_ref, wu2t_ref, bu2_ref,
               g_ref, bt_ref, out_ref):
    seg = accs_ref[0] + accs_ref[1]
    cnt = (accc_ref[0] + accc_ref[1])[:, :1]
    agg_sum = jnp.dot(seg, w2t_ref[...],
                      preferred_element_type=jnp.float32) + cnt * b2_ref[...]
    agg = agg_sum / jnp.maximum(cnt, 1.0)
    hd = hd_ref[...]
    pre = (jnp.dot(hd, wu1a_ref[...], preferred_element_type=jnp.float32)
           + jnp.dot(agg, wu1b_ref[...], preferred_element_type=jnp.float32)
           + bu1_ref[...])
    dmid = _silu(pre)
    delta = jnp.dot(dmid, wu2t_ref[...],
                    preferred_element_type=jnp.float32) + bu2_ref[...]
    y = hd + delta
    mean = jnp.mean(y, axis=1, keepdims=True)
    d = y - mean
    var = jnp.mean(d * d, axis=1, keepdims=True)
    out_ref[...] = d * lax.rsqrt(var + 1e-5) * g_ref[...] + bt_ref[...]


def kernel(h_src, h_dst, edge_index, edge_attr, n_dst,
           W_msg1, b_msg1, W_msg2, b_msg2,
           W_upd1, b_upd1, W_upd2, b_upd2, gamma, beta):
    n_nodes = h_dst.shape[0]
    n_edges = edge_attr.shape[0]
    ef = edge_attr.shape[1]
    epw = n_edges // NW
    assert epw * NW == n_edges

    cbe = CH * BE
    epw_pad = -(-epw // cbe) * cbe
    nchunk = epw_pad // cbe
    stripe = -(-(-(-n_nodes // NSUB)) // 8) * 8
    n_pad = stripe * NSUB

    w1t = W_msg1.T
    w1a = w1t[:HID]
    w1b = w1t[HID:2 * HID]
    w1c = w1t[2 * HID:]

    f32 = jnp.float32
    hs_p = jnp.pad(h_src, ((0, n_pad - n_nodes), (0, 0)))
    hd_p = jnp.pad(h_dst, ((0, n_pad - n_nodes), (0, 0)))

    row_n = n_pad // 8
    p_src, p_dst = pl.pallas_call(
        _pre_node_body,
        grid=(8,),
        in_specs=[
            pl.BlockSpec((row_n, HID), lambda i: (i, 0)),
            pl.BlockSpec((row_n, HID), lambda i: (i, 0)),
            pl.BlockSpec((HID, HID), lambda i: (0, 0)),
            pl.BlockSpec((HID, HID), lambda i: (0, 0)),
            pl.BlockSpec((1, HID), lambda i: (0, 0)),
        ],
        out_specs=[
            pl.BlockSpec((row_n, HID), lambda i: (i, 0)),
            pl.BlockSpec((row_n, HID), lambda i: (i, 0)),
        ],
        out_shape=[
            jax.ShapeDtypeStruct((n_pad, HID), f32),
            jax.ShapeDtypeStruct((n_pad, HID), f32),
        ],
    )(hs_p, hd_p, w1a, w1b, b_msg1.reshape(1, HID))

    ne_pad = NW * epw_pad
    ea_p = jnp.pad(edge_attr.reshape(NW, epw, ef),
                   ((0, 0), (0, epw_pad - epw), (0, 0))).reshape(ne_pad, ef)
    idx_pad = jnp.int32(n_pad - 1)
    src_r = jnp.pad(edge_index[0].astype(jnp.int32).reshape(NW, epw),
                    ((0, 0), (0, epw_pad - epw)),
                    constant_values=idx_pad).reshape(NW, nchunk, CH, BE)
    dst_r = jnp.pad(edge_index[1].astype(jnp.int32).reshape(NW, epw),
                    ((0, 0), (0, epw_pad - epw)),
                    constant_values=idx_pad).reshape(NW, nchunk, CH, BE)

    row_e = 4096
    grid_e = ne_pad // row_e
    assert grid_e * row_e == ne_pad
    e_proj = pl.pallas_call(
        _pre_edge_body,
        grid=(grid_e,),
        in_specs=[
            pl.BlockSpec((row_e, ef), lambda i: (i, 0)),
            pl.BlockSpec((ef, HID), lambda i: (0, 0)),
        ],
        out_specs=pl.BlockSpec((row_e, HID), lambda i: (i, 0)),
        out_shape=jax.ShapeDtypeStruct((ne_pad, HID), f32),
    )(ea_p, w1c)

    sc_edge = functools.partial(
        pl.kernel,
        out_type=jax.ShapeDtypeStruct((NCORE, n_pad, HID), f32),
        mesh=plsc.VectorSubcoreMesh(core_axis_name="c", subcore_axis_name="s"),
        scratch_types=[
            pltpu.MemorySpace.VMEM((CH, BE), jnp.int32),
            pltpu.MemorySpace.VMEM((CH, BE), jnp.int32),
            pltpu.MemorySpace.VMEM((BE, HID), f32),
            pltpu.MemorySpace.VMEM((BE, HID), f32),
            pltpu.MemorySpace.VMEM_SHARED((n_pad, HID), f32),
            pltpu.SemaphoreType.DMA,
            pltpu.SemaphoreType.DMA,
            pltpu.SemaphoreType.DMA,
        ],
    )(functools.partial(_sc_edge_body, n_pad, nchunk))

    acc_s = sc_edge(p_src, p_dst, e_proj, src_r, dst_r)

    sc_count = functools.partial(
        pl.kernel,
        out_type=jax.ShapeDtypeStruct((NCORE, n_pad, HID), f32),
        mesh=plsc.VectorSubcoreMesh(core_axis_name="c", subcore_axis_name="s"),
        scratch_types=[
            pltpu.MemorySpace.VMEM((CH, BE), jnp.int32),
            pltpu.MemorySpace.VMEM((BE, HID), f32),
            pltpu.MemorySpace.VMEM_SHARED((n_pad, HID), f32),
        ],
    )(functools.partial(_sc_count_body, n_pad, nchunk))

    acc_c = sc_count(dst_r)

    wu1t = W_upd1.T
    row_p = n_nodes // 10
    out = pl.pallas_call(
        _post_body,
        grid=(10,),
        in_specs=[
            pl.BlockSpec((NCORE, row_p, HID), lambda i: (0, i, 0)),
            pl.BlockSpec((NCORE, row_p, HID), lambda i: (0, i, 0)),
            pl.BlockSpec((row_p, HID), lambda i: (i, 0)),
            pl.BlockSpec((HID, HID), lambda i: (0, 0)),
            pl.BlockSpec((1, HID), lambda i: (0, 0)),
            pl.BlockSpec((HID, HID), lambda i: (0, 0)),
            pl.BlockSpec((HID, HID), lambda i: (0, 0)),
            pl.BlockSpec((1, HID), lambda i: (0, 0)),
            pl.BlockSpec((HID, HID), lambda i: (0, 0)),
            pl.BlockSpec((1, HID), lambda i: (0, 0)),
            pl.BlockSpec((1, HID), lambda i: (0, 0)),
            pl.BlockSpec((1, HID), lambda i: (0, 0)),
        ],
        out_specs=pl.BlockSpec((row_p, HID), lambda i: (i, 0)),
        out_shape=jax.ShapeDtypeStruct((n_nodes, HID), f32),
    )(acc_s, acc_c, h_dst, W_msg2.T, b_msg2.reshape(1, HID),
      wu1t[:HID], wu1t[HID:], b_upd1.reshape(1, HID),
      W_upd2.T, b_upd2.reshape(1, HID),
      gamma.reshape(1, HID), beta.reshape(1, HID))
    return out


# speedup vs baseline: 2.6136x; 1.0161x over previous
import functools

import jax
import jax.numpy as jnp
from jax import lax
from jax.experimental import pallas as pl
from jax.experimental.pallas import tpu as pltpu
from jax.experimental.pallas import tpu_sc as plsc

HID = 128
NCORE = 2
NSUB = 16
NW = NCORE * NSUB
BE = 128
CH = 8


def _silu(x):
    return x / (1.0 + jnp.exp(-x))


def _pre_node_body(hs_ref, hd_ref, w1a_ref, w1b_ref, b1_ref, ps_ref, pd_ref):
    ps_ref[...] = jnp.dot(hs_ref[...], w1a_ref[...],
                          preferred_element_type=jnp.float32)
    pd_ref[...] = jnp.dot(hd_ref[...], w1b_ref[...],
                          preferred_element_type=jnp.float32) + b1_ref[...]


def _pre_edge_body(ea_ref, w1c_ref, e_ref):
    e_ref[...] = jnp.dot(ea_ref[...], w1c_ref[...],
                         preferred_element_type=jnp.float32)


def _zero_fill(a_v, ref, s, stripe):
    nrep = stripe // BE
    rem = stripe % BE
    for t in range(nrep):
        pltpu.sync_copy(a_v, ref.at[pl.ds(s * stripe + t * BE, BE)])
    if rem:
        pltpu.sync_copy(a_v.at[pl.ds(0, rem)],
                        ref.at[pl.ds(s * stripe + nrep * BE, rem)])


def _sc_edge_body(n_pad, nchunk, epw_real, ps_hbm, pd_hbm, e_hbm, src_hbm, dst_hbm,
                  outs_hbm,
                  si_v, di_v, a_v, b_v,
                  acc_s, sem_a, sem_b, sem_c):
    c = lax.axis_index("c")
    s = lax.axis_index("s")
    wid = c * NSUB + s
    stripe = n_pad // NSUB
    epw = nchunk * CH * BE
    imax = jnp.int32(n_pad - 1)

    zeros = jnp.zeros((16,), jnp.float32)

    def _zrow(r, carry):
        for cc in range(8):
            a_v[r, pl.ds(cc * 16, 16)] = zeros
        return carry

    lax.fori_loop(0, BE, _zrow, 0)
    _zero_fill(a_v, acc_s, s, stripe)
    plsc.subcore_barrier()

    def _mkclamp(idx_v):
        def _clamp(i, carry):
            r = i >> 3
            cc = (i & 7) * 16
            v = idx_v[r, pl.ds(cc, 16)]
            idx_v[r, pl.ds(cc, 16)] = jnp.minimum(jnp.maximum(v, 0), imax)
            return carry
        return _clamp

    def _chunk(t, carry):
        pltpu.sync_copy(src_hbm.at[wid, t], si_v)
        pltpu.sync_copy(dst_hbm.at[wid, t], di_v)
        lax.fori_loop(0, CH * 8, _mkclamp(si_v), 0)
        lax.fori_loop(0, CH * 8, _mkclamp(di_v), 0)

        def _addpass(base):
            def f(i, inner2):
                r = base + i * 4
                for rr in range(4):
                    for cc in range(8):
                        sl = pl.ds(cc * 16, 16)
                        a_v[r + rr, sl] = a_v[r + rr, sl] + b_v[r + rr, sl]
                return inner2
            lax.fori_loop(0, BE // 8, f, 0)

        def _silupass(base):
            def f(i, inner2):
                r = base + i * 4
                for rr in range(4):
                    for cc in range(8):
                        sl = pl.ds(cc * 16, 16)
                        tt = a_v[r + rr, sl] + b_v[r + rr, sl]
                        a_v[r + rr, sl] = tt / (1.0 + jnp.exp(-tt))
                return inner2
            lax.fori_loop(0, BE // 8, f, 0)

        H = BE // 2

        def _block(j, inner):
            rowb = wid * epw_real + (t * CH + j) * BE
            cps0 = pltpu.async_copy(ps_hbm.at[si_v.at[j, pl.ds(0, H)]],
                                    a_v.at[pl.ds(0, H)], sem_a)
            cpe0 = pltpu.async_copy(e_hbm.at[pl.ds(rowb, H)],
                                    b_v.at[pl.ds(0, H)], sem_b)
            cps1 = pltpu.async_copy(ps_hbm.at[si_v.at[j, pl.ds(H, H)]],
                                    a_v.at[pl.ds(H, H)], sem_a)
            cpe1 = pltpu.async_copy(e_hbm.at[pl.ds(rowb + H, H)],
                                    b_v.at[pl.ds(H, H)], sem_b)
            cps0.wait()
            cpe0.wait()
            _addpass(0)
            cpd0 = pltpu.async_copy(pd_hbm.at[di_v.at[j, pl.ds(0, H)]],
                                    b_v.at[pl.ds(0, H)], sem_c)
            cps1.wait()
            cpe1.wait()
            _addpass(H)
            cpd1 = pltpu.async_copy(pd_hbm.at[di_v.at[j, pl.ds(H, H)]],
                                    b_v.at[pl.ds(H, H)], sem_c)
            cpd0.wait()
            _silupass(0)
            cpd1.wait()
            _silupass(H)
            pltpu.sync_copy(a_v, acc_s.at[di_v.at[j]], add=True)
            return inner

        lax.fori_loop(0, CH, _block, 0)
        return carry

    lax.fori_loop(0, nchunk, _chunk, 0)
    plsc.subcore_barrier()

    base = s * stripe
    pltpu.sync_copy(acc_s.at[pl.ds(base, stripe)],
                    outs_hbm.at[c, pl.ds(base, stripe)])


def _sc_count_body(n_pad, nchunk, dst_hbm, outc_hbm,
                   idx_v, one_v, acc_c):
    c = lax.axis_index("c")
    s = lax.axis_index("s")
    wid = c * NSUB + s
    stripe = n_pad // NSUB
    imax = jnp.int32(n_pad - 1)

    zeros = jnp.zeros((16,), jnp.float32)

    def _zrow(r, carry):
        for cc in range(8):
            one_v[r, pl.ds(cc * 16, 16)] = zeros
        return carry

    lax.fori_loop(0, BE, _zrow, 0)
    _zero_fill(one_v, acc_c, s, stripe)

    ones = jnp.ones((16,), jnp.float32)

    def _orow(r, carry):
        one_v[r, pl.ds(0, 16)] = ones
        return carry

    lax.fori_loop(0, BE, _orow, 0)
    plsc.subcore_barrier()

    def _clamp(i, carry):
        r = i >> 3
        cc = (i & 7) * 16
        v = idx_v[r, pl.ds(cc, 16)]
        idx_v[r, pl.ds(cc, 16)] = jnp.minimum(jnp.maximum(v, 0), imax)
        return carry

    def _chunk(t, carry):
        pltpu.sync_copy(dst_hbm.at[wid, t], idx_v)
        lax.fori_loop(0, CH * 8, _clamp, 0)

        def _block(j, inner):
            pltpu.sync_copy(one_v, acc_c.at[idx_v.at[j]], add=True)
            return inner

        lax.fori_loop(0, CH, _block, 0)
        return carry

    lax.fori_loop(0, nchunk, _chunk, 0)
    plsc.subcore_barrier()

    base = s * stripe
    pltpu.sync_copy(acc_c.at[pl.ds(base, stripe)],
                    outc_hbm.at[c, pl.ds(base, stripe)])


def _post_body(accs_ref, accc_ref, hd_ref, w2t_ref, b2_ref,
               wu1a_ref, wu1b_ref, bu1_ref, wu2t_ref, bu2_ref,
               g_ref, bt_ref, out_ref):
    seg = accs_ref[0] + accs_ref[1]
    cnt = (accc_ref[0] + accc_ref[1])[:, :1]
    agg_sum = jnp.dot(seg, w2t_ref[...],
                      preferred_element_type=jnp.float32) + cnt * b2_ref[...]
    agg = agg_sum / jnp.maximum(cnt, 1.0)
    hd = hd_ref[...]
    pre = (jnp.dot(hd, wu1a_ref[...], preferred_element_type=jnp.float32)
           + jnp.dot(agg, wu1b_ref[...], preferred_element_type=jnp.float32)
           + bu1_ref[...])
    dmid = _silu(pre)
    delta = jnp.dot(dmid, wu2t_ref[...],
                    preferred_element_type=jnp.float32) + bu2_ref[...]
    y = hd + delta
    mean = jnp.mean(y, axis=1, keepdims=True)
    d = y - mean
    var = jnp.mean(d * d, axis=1, keepdims=True)
    out_ref[...] = d * lax.rsqrt(var + 1e-5) * g_ref[...] + bt_ref[...]


def kernel(h_src, h_dst, edge_index, edge_attr, n_dst,
           W_msg1, b_msg1, W_msg2, b_msg2,
           W_upd1, b_upd1, W_upd2, b_upd2, gamma, beta):
    n_nodes = h_dst.shape[0]
    n_edges = edge_attr.shape[0]
    ef = edge_attr.shape[1]
    epw = n_edges // NW
    assert epw * NW == n_edges

    cbe = CH * BE
    epw_pad = -(-epw // cbe) * cbe
    nchunk = epw_pad // cbe
    stripe = -(-(-(-n_nodes // NSUB)) // 8) * 8
    n_pad = stripe * NSUB

    w1t = W_msg1.T
    w1a = w1t[:HID]
    w1b = w1t[HID:2 * HID]
    w1c = w1t[2 * HID:]

    f32 = jnp.float32
    hs_p = jnp.pad(h_src, ((0, n_pad - n_nodes), (0, 0)))
    hd_p = jnp.pad(h_dst, ((0, n_pad - n_nodes), (0, 0)))

    row_n = n_pad // 8
    p_src, p_dst = pl.pallas_call(
        _pre_node_body,
        grid=(8,),
        in_specs=[
            pl.BlockSpec((row_n, HID), lambda i: (i, 0)),
            pl.BlockSpec((row_n, HID), lambda i: (i, 0)),
            pl.BlockSpec((HID, HID), lambda i: (0, 0)),
            pl.BlockSpec((HID, HID), lambda i: (0, 0)),
            pl.BlockSpec((1, HID), lambda i: (0, 0)),
        ],
        out_specs=[
            pl.BlockSpec((row_n, HID), lambda i: (i, 0)),
            pl.BlockSpec((row_n, HID), lambda i: (i, 0)),
        ],
        out_shape=[
            jax.ShapeDtypeStruct((n_pad, HID), f32),
            jax.ShapeDtypeStruct((n_pad, HID), f32),
        ],
    )(hs_p, hd_p, w1a, w1b, b_msg1.reshape(1, HID))

    ne_pad = NW * epw_pad
    ea_p = jnp.pad(edge_attr, ((0, ne_pad - n_edges), (0, 0)))
    idx_pad = jnp.int32(n_pad - 1)
    src_r = jnp.pad(edge_index[0].astype(jnp.int32).reshape(NW, epw),
                    ((0, 0), (0, epw_pad - epw)),
                    constant_values=idx_pad).reshape(NW, nchunk, CH, BE)
    dst_r = jnp.pad(edge_index[1].astype(jnp.int32).reshape(NW, epw),
                    ((0, 0), (0, epw_pad - epw)),
                    constant_values=idx_pad).reshape(NW, nchunk, CH, BE)

    row_e = 4096
    grid_e = ne_pad // row_e
    assert grid_e * row_e == ne_pad
    e_proj = pl.pallas_call(
        _pre_edge_body,
        grid=(grid_e,),
        in_specs=[
            pl.BlockSpec((row_e, ef), lambda i: (i, 0)),
            pl.BlockSpec((ef, HID), lambda i: (0, 0)),
        ],
        out_specs=pl.BlockSpec((row_e, HID), lambda i: (i, 0)),
        out_shape=jax.ShapeDtypeStruct((ne_pad, HID), f32),
    )(ea_p, w1c)

    sc_edge = functools.partial(
        pl.kernel,
        out_type=jax.ShapeDtypeStruct((NCORE, n_pad, HID), f32),
        mesh=plsc.VectorSubcoreMesh(core_axis_name="c", subcore_axis_name="s"),
        scratch_types=[
            pltpu.MemorySpace.VMEM((CH, BE), jnp.int32),
            pltpu.MemorySpace.VMEM((CH, BE), jnp.int32),
            pltpu.MemorySpace.VMEM((BE, HID), f32),
            pltpu.MemorySpace.VMEM((BE, HID), f32),
            pltpu.MemorySpace.VMEM_SHARED((n_pad, HID), f32),
            pltpu.SemaphoreType.DMA,
            pltpu.SemaphoreType.DMA,
            pltpu.SemaphoreType.DMA,
        ],
    )(functools.partial(_sc_edge_body, n_pad, nchunk, epw))

    acc_s = sc_edge(p_src, p_dst, e_proj, src_r, dst_r)

    sc_count = functools.partial(
        pl.kernel,
        out_type=jax.ShapeDtypeStruct((NCORE, n_pad, HID), f32),
        mesh=plsc.VectorSubcoreMesh(core_axis_name="c", subcore_axis_name="s"),
        scratch_types=[
            pltpu.MemorySpace.VMEM((CH, BE), jnp.int32),
            pltpu.MemorySpace.VMEM((BE, HID), f32),
            pltpu.MemorySpace.VMEM_SHARED((n_pad, HID), f32),
        ],
    )(functools.partial(_sc_count_body, n_pad, nchunk))

    acc_c = sc_count(dst_r)

    wu1t = W_upd1.T
    row_p = n_nodes // 10
    out = pl.pallas_call(
        _post_body,
        grid=(10,),
        in_specs=[
            pl.BlockSpec((NCORE, row_p, HID), lambda i: (0, i, 0)),
            pl.BlockSpec((NCORE, row_p, HID), lambda i: (0, i, 0)),
            pl.BlockSpec((row_p, HID), lambda i: (i, 0)),
            pl.BlockSpec((HID, HID), lambda i: (0, 0)),
            pl.BlockSpec((1, HID), lambda i: (0, 0)),
            pl.BlockSpec((HID, HID), lambda i: (0, 0)),
            pl.BlockSpec((HID, HID), lambda i: (0, 0)),
            pl.BlockSpec((1, HID), lambda i: (0, 0)),
            pl.BlockSpec((HID, HID), lambda i: (0, 0)),
            pl.BlockSpec((1, HID), lambda i: (0, 0)),
            pl.BlockSpec((1, HID), lambda i: (0, 0)),
            pl.BlockSpec((1, HID), lambda i: (0, 0)),
        ],
        out_specs=pl.BlockSpec((row_p, HID), lambda i: (i, 0)),
        out_shape=jax.ShapeDtypeStruct((n_nodes, HID), f32),
    )(acc_s, acc_c, h_dst, W_msg2.T, b_msg2.reshape(1, HID),
      wu1t[:HID], wu1t[HID:], b_upd1.reshape(1, HID),
      W_upd2.T, b_upd2.reshape(1, HID),
      gamma.reshape(1, HID), beta.reshape(1, HID))
    return out


# speedup vs baseline: 2.7277x; 1.0436x over previous
import functools

import jax
import jax.numpy as jnp
from jax import lax
from jax.experimental import pallas as pl
from jax.experimental.pallas import tpu as pltpu
from jax.experimental.pallas import tpu_sc as plsc

HID = 128
NCORE = 2
NSUB = 16
NW = NCORE * NSUB
BE = 128
CH = 8


def _silu(x):
    return x / (1.0 + jnp.exp(-x))


def _pre_node_body(hs_ref, hd_ref, w1a_ref, w1b_ref, b1_ref, ps_ref, pd_ref):
    ps_ref[...] = jnp.dot(hs_ref[...], w1a_ref[...],
                          preferred_element_type=jnp.float32)
    pd_ref[...] = jnp.dot(hd_ref[...], w1b_ref[...],
                          preferred_element_type=jnp.float32) + b1_ref[...]


def _pre_edge_body(ea_ref, w1c_ref, e_ref):
    e_ref[...] = jnp.dot(ea_ref[...], w1c_ref[...],
                         preferred_element_type=jnp.float32)


def _zero_fill(a_v, ref, s, stripe):
    nrep = stripe // BE
    rem = stripe % BE
    for t in range(nrep):
        pltpu.sync_copy(a_v, ref.at[pl.ds(s * stripe + t * BE, BE)])
    if rem:
        pltpu.sync_copy(a_v.at[pl.ds(0, rem)],
                        ref.at[pl.ds(s * stripe + nrep * BE, rem)])


def _sc_edge_body(n_pad, nchunk, epw_real, ps_hbm, pd_hbm, e_hbm, src_hbm, dst_hbm,
                  outs_hbm,
                  si_v, di_v, a_v, b_v,
                  acc_s, sem_a, sem_b, sem_c):
    c = lax.axis_index("c")
    s = lax.axis_index("s")
    wid = c * NSUB + s
    stripe = n_pad // NSUB
    epw = nchunk * CH * BE
    imax = jnp.int32(n_pad - 1)

    zeros = jnp.zeros((16,), jnp.float32)

    def _zrow(r, carry):
        for cc in range(8):
            a_v[r, pl.ds(cc * 16, 16)] = zeros
        return carry

    lax.fori_loop(0, BE, _zrow, 0)
    _zero_fill(a_v, acc_s, s, stripe)
    plsc.subcore_barrier()

    def _mkclamp(idx_v):
        def _clamp(i, carry):
            r = i >> 3
            cc = (i & 7) * 16
            v = idx_v[r, pl.ds(cc, 16)]
            idx_v[r, pl.ds(cc, 16)] = jnp.minimum(jnp.maximum(v, 0), imax)
            return carry
        return _clamp

    def _chunk(t, carry):
        pltpu.sync_copy(src_hbm.at[wid, t], si_v)
        pltpu.sync_copy(dst_hbm.at[wid, t], di_v)
        lax.fori_loop(0, CH * 8, _mkclamp(si_v), 0)
        lax.fori_loop(0, CH * 8, _mkclamp(di_v), 0)

        def _addpass(base):
            def f(i, inner2):
                r = base + i * 4
                for rr in range(4):
                    for cc in range(8):
                        sl = pl.ds(cc * 16, 16)
                        a_v[r + rr, sl] = a_v[r + rr, sl] + b_v[r + rr, sl]
                return inner2
            lax.fori_loop(0, BE // 8, f, 0)

        def _silupass(base):
            def f(i, inner2):
                r = base + i * 4
                for rr in range(4):
                    for cc in range(8):
                        sl = pl.ds(cc * 16, 16)
                        tt = a_v[r + rr, sl] + b_v[r + rr, sl]
                        b_v[r + rr, sl] = tt / (1.0 + jnp.exp(-tt))
                return inner2
            lax.fori_loop(0, BE // 8, f, 0)

        H = BE // 2

        pltpu.async_copy(ps_hbm.at[si_v.at[0, pl.ds(0, H)]],
                         a_v.at[pl.ds(0, H)], sem_a)
        pltpu.async_copy(ps_hbm.at[si_v.at[0, pl.ds(H, H)]],
                         a_v.at[pl.ds(H, H)], sem_a)

        def _block(j, inner):
            rowb = wid * epw_real + (t * CH + j) * BE
            cpe0 = pltpu.async_copy(e_hbm.at[pl.ds(rowb, H)],
                                    b_v.at[pl.ds(0, H)], sem_b)
            cpe1 = pltpu.async_copy(e_hbm.at[pl.ds(rowb + H, H)],
                                    b_v.at[pl.ds(H, H)], sem_b)
            pltpu.make_async_copy(ps_hbm.at[si_v.at[j, pl.ds(0, H)]],
                                  a_v.at[pl.ds(0, H)], sem_a).wait()
            cpe0.wait()
            _addpass(0)
            cpd0 = pltpu.async_copy(pd_hbm.at[di_v.at[j, pl.ds(0, H)]],
                                    b_v.at[pl.ds(0, H)], sem_c)
            pltpu.make_async_copy(ps_hbm.at[si_v.at[j, pl.ds(H, H)]],
                                  a_v.at[pl.ds(H, H)], sem_a).wait()
            cpe1.wait()
            _addpass(H)
            cpd1 = pltpu.async_copy(pd_hbm.at[di_v.at[j, pl.ds(H, H)]],
                                    b_v.at[pl.ds(H, H)], sem_c)
            cpd0.wait()
            _silupass(0)
            cpd1.wait()
            _silupass(H)

            @pl.when(j < CH - 1)
            def _prefetch():
                pltpu.async_copy(ps_hbm.at[si_v.at[j + 1, pl.ds(0, H)]],
                                 a_v.at[pl.ds(0, H)], sem_a)
                pltpu.async_copy(ps_hbm.at[si_v.at[j + 1, pl.ds(H, H)]],
                                 a_v.at[pl.ds(H, H)], sem_a)

            pltpu.sync_copy(b_v, acc_s.at[di_v.at[j]], add=True)
            return inner

        lax.fori_loop(0, CH, _block, 0)
        return carry

    lax.fori_loop(0, nchunk, _chunk, 0)
    plsc.subcore_barrier()

    base = s * stripe
    pltpu.sync_copy(acc_s.at[pl.ds(base, stripe)],
                    outs_hbm.at[c, pl.ds(base, stripe)])


def _sc_count_body(n_pad, nchunk, dst_hbm, outc_hbm,
                   idx_v, one_v, acc_c):
    c = lax.axis_index("c")
    s = lax.axis_index("s")
    wid = c * NSUB + s
    stripe = n_pad // NSUB
    imax = jnp.int32(n_pad - 1)

    zeros = jnp.zeros((16,), jnp.float32)

    def _zrow(r, carry):
        for cc in range(8):
            one_v[r, pl.ds(cc * 16, 16)] = zeros
        return carry

    lax.fori_loop(0, BE, _zrow, 0)
    _zero_fill(one_v, acc_c, s, stripe)

    ones = jnp.ones((16,), jnp.float32)

    def _orow(r, carry):
        one_v[r, pl.ds(0, 16)] = ones
        return carry

    lax.fori_loop(0, BE, _orow, 0)
    plsc.subcore_barrier()

    def _clamp(i, carry):
        r = i >> 3
        cc = (i & 7) * 16
        v = idx_v[r, pl.ds(cc, 16)]
        idx_v[r, pl.ds(cc, 16)] = jnp.minimum(jnp.maximum(v, 0), imax)
        return carry

    def _chunk(t, carry):
        pltpu.sync_copy(dst_hbm.at[wid, t], idx_v)
        lax.fori_loop(0, CH * 8, _clamp, 0)

        def _block(j, inner):
            pltpu.sync_copy(one_v, acc_c.at[idx_v.at[j]], add=True)
            return inner

        lax.fori_loop(0, CH, _block, 0)
        return carry

    lax.fori_loop(0, nchunk, _chunk, 0)
    plsc.subcore_barrier()

    base = s * stripe
    pltpu.sync_copy(acc_c.at[pl.ds(base, stripe)],
                    outc_hbm.at[c, pl.ds(base, stripe)])


def _post_body(accs_ref, accc_ref, hd_ref, w2t_ref, b2_ref,
               wu1a_ref, wu1b_ref, bu1_ref, wu2t_ref, bu2_ref,
               g_ref, bt_ref, out_ref):
    seg = accs_ref[0] + accs_ref[1]
    cnt = (accc_ref[0] + accc_ref[1])[:, :1]
    agg_sum = jnp.dot(seg, w2t_ref[...],
                      preferred_element_type=jnp.float32) + cnt * b2_ref[...]
    agg = agg_sum / jnp.maximum(cnt, 1.0)
    hd = hd_ref[...]
    pre = (jnp.dot(hd, wu1a_ref[...], preferred_element_type=jnp.float32)
           + jnp.dot(agg, wu1b_ref[...], preferred_element_type=jnp.float32)
           + bu1_ref[...])
    dmid = _silu(pre)
    delta = jnp.dot(dmid, wu2t_ref[...],
                    preferred_element_type=jnp.float32) + bu2_ref[...]
    y = hd + delta
    mean = jnp.mean(y, axis=1, keepdims=True)
    d = y - mean
    var = jnp.mean(d * d, axis=1, keepdims=True)
    out_ref[...] = d * lax.rsqrt(var + 1e-5) * g_ref[...] + bt_ref[...]


def kernel(h_src, h_dst, edge_index, edge_attr, n_dst,
           W_msg1, b_msg1, W_msg2, b_msg2,
           W_upd1, b_upd1, W_upd2, b_upd2, gamma, beta):
    n_nodes = h_dst.shape[0]
    n_edges = edge_attr.shape[0]
    ef = edge_attr.shape[1]
    epw = n_edges // NW
    assert epw * NW == n_edges

    cbe = CH * BE
    epw_pad = -(-epw // cbe) * cbe
    nchunk = epw_pad // cbe
    stripe = -(-(-(-n_nodes // NSUB)) // 8) * 8
    n_pad = stripe * NSUB

    w1t = W_msg1.T
    w1a = w1t[:HID]
    w1b = w1t[HID:2 * HID]
    w1c = w1t[2 * HID:]

    f32 = jnp.float32
    hs_p = jnp.pad(h_src, ((0, n_pad - n_nodes), (0, 0)))
    hd_p = jnp.pad(h_dst, ((0, n_pad - n_nodes), (0, 0)))

    row_n = n_pad // 8
    p_src, p_dst = pl.pallas_call(
        _pre_node_body,
        grid=(8,),
        in_specs=[
            pl.BlockSpec((row_n, HID), lambda i: (i, 0)),
            pl.BlockSpec((row_n, HID), lambda i: (i, 0)),
            pl.BlockSpec((HID, HID), lambda i: (0, 0)),
            pl.BlockSpec((HID, HID), lambda i: (0, 0)),
            pl.BlockSpec((1, HID), lambda i: (0, 0)),
        ],
        out_specs=[
            pl.BlockSpec((row_n, HID), lambda i: (i, 0)),
            pl.BlockSpec((row_n, HID), lambda i: (i, 0)),
        ],
        out_shape=[
            jax.ShapeDtypeStruct((n_pad, HID), f32),
            jax.ShapeDtypeStruct((n_pad, HID), f32),
        ],
    )(hs_p, hd_p, w1a, w1b, b_msg1.reshape(1, HID))

    ne_pad = NW * epw_pad
    ea_p = jnp.pad(edge_attr, ((0, ne_pad - n_edges), (0, 0)))
    idx_pad = jnp.int32(n_pad - 1)
    src_r = jnp.pad(edge_index[0].astype(jnp.int32).reshape(NW, epw),
                    ((0, 0), (0, epw_pad - epw)),
                    constant_values=idx_pad).reshape(NW, nchunk, CH, BE)
    dst_r = jnp.pad(edge_index[1].astype(jnp.int32).reshape(NW, epw),
                    ((0, 0), (0, epw_pad - epw)),
                    constant_values=idx_pad).reshape(NW, nchunk, CH, BE)

    row_e = 4096
    grid_e = ne_pad // row_e
    assert grid_e * row_e == ne_pad
    e_proj = pl.pallas_call(
        _pre_edge_body,
        grid=(grid_e,),
        in_specs=[
            pl.BlockSpec((row_e, ef), lambda i: (i, 0)),
            pl.BlockSpec((ef, HID), lambda i: (0, 0)),
        ],
        out_specs=pl.BlockSpec((row_e, HID), lambda i: (i, 0)),
        out_shape=jax.ShapeDtypeStruct((ne_pad, HID), f32),
    )(ea_p, w1c)

    sc_edge = functools.partial(
        pl.kernel,
        out_type=jax.ShapeDtypeStruct((NCORE, n_pad, HID), f32),
        mesh=plsc.VectorSubcoreMesh(core_axis_name="c", subcore_axis_name="s"),
        scratch_types=[
            pltpu.MemorySpace.VMEM((CH, BE), jnp.int32),
            pltpu.MemorySpace.VMEM((CH, BE), jnp.int32),
            pltpu.MemorySpace.VMEM((BE, HID), f32),
            pltpu.MemorySpace.VMEM((BE, HID), f32),
            pltpu.MemorySpace.VMEM_SHARED((n_pad, HID), f32),
            pltpu.SemaphoreType.DMA,
            pltpu.SemaphoreType.DMA,
            pltpu.SemaphoreType.DMA,
        ],
    )(functools.partial(_sc_edge_body, n_pad, nchunk, epw))

    acc_s = sc_edge(p_src, p_dst, e_proj, src_r, dst_r)

    sc_count = functools.partial(
        pl.kernel,
        out_type=jax.ShapeDtypeStruct((NCORE, n_pad, HID), f32),
        mesh=plsc.VectorSubcoreMesh(core_axis_name="c", subcore_axis_name="s"),
        scratch_types=[
            pltpu.MemorySpace.VMEM((CH, BE), jnp.int32),
            pltpu.MemorySpace.VMEM((BE, HID), f32),
            pltpu.MemorySpace.VMEM_SHARED((n_pad, HID), f32),
        ],
    )(functools.partial(_sc_count_body, n_pad, nchunk))

    acc_c = sc_count(dst_r)

    wu1t = W_upd1.T
    row_p = n_nodes // 10
    out = pl.pallas_call(
        _post_body,
        grid=(10,),
        in_specs=[
            pl.BlockSpec((NCORE, row_p, HID), lambda i: (0, i, 0)),
            pl.BlockSpec((NCORE, row_p, HID), lambda i: (0, i, 0)),
            pl.BlockSpec((row_p, HID), lambda i: (i, 0)),
            pl.BlockSpec((HID, HID), lambda i: (0, 0)),
            pl.BlockSpec((1, HID), lambda i: (0, 0)),
            pl.BlockSpec((HID, HID), lambda i: (0, 0)),
            pl.BlockSpec((HID, HID), lambda i: (0, 0)),
            pl.BlockSpec((1, HID), lambda i: (0, 0)),
            pl.BlockSpec((HID, HID), lambda i: (0, 0)),
            pl.BlockSpec((1, HID), lambda i: (0, 0)),
            pl.BlockSpec((1, HID), lambda i: (0, 0)),
            pl.BlockSpec((1, HID), lambda i: (0, 0)),
        ],
        out_specs=pl.BlockSpec((row_p, HID), lambda i: (i, 0)),
        out_shape=jax.ShapeDtypeStruct((n_nodes, HID), f32),
    )(acc_s, acc_c, h_dst, W_msg2.T, b_msg2.reshape(1, HID),
      wu1t[:HID], wu1t[HID:], b_upd1.reshape(1, HID),
      W_upd2.T, b_upd2.reshape(1, HID),
      gamma.reshape(1, HID), beta.reshape(1, HID))
    return out


# speedup vs baseline: 2.7309x; 1.0012x over previous
import functools

import jax
import jax.numpy as jnp
from jax import lax
from jax.experimental import pallas as pl
from jax.experimental.pallas import tpu as pltpu
from jax.experimental.pallas import tpu_sc as plsc

HID = 128
NCORE = 2
NSUB = 16
NW = NCORE * NSUB
BE = 128
CH = 8


def _silu(x):
    return x / (1.0 + jnp.exp(-x))


def _pre_node_body(hs_ref, hd_ref, w1a_ref, w1b_ref, b1_ref, ps_ref, pd_ref):
    ps_ref[...] = jnp.dot(hs_ref[...], w1a_ref[...],
                          preferred_element_type=jnp.float32)
    pd_ref[...] = jnp.dot(hd_ref[...], w1b_ref[...],
                          preferred_element_type=jnp.float32) + b1_ref[...]


def _pre_edge_body(ea_ref, w1c_ref, e_ref):
    e_ref[...] = jnp.dot(ea_ref[...], w1c_ref[...],
                         preferred_element_type=jnp.float32)


def _zero_fill(a_v, ref, s, stripe):
    nrep = stripe // BE
    rem = stripe % BE
    for t in range(nrep):
        pltpu.sync_copy(a_v, ref.at[pl.ds(s * stripe + t * BE, BE)])
    if rem:
        pltpu.sync_copy(a_v.at[pl.ds(0, rem)],
                        ref.at[pl.ds(s * stripe + nrep * BE, rem)])


def _sc_edge_body(n_pad, nchunk, epw_real, ps_hbm, pd_hbm, e_hbm, src_hbm, dst_hbm,
                  outs_hbm,
                  si_v, di_v, a_v, b_v,
                  acc_s, sem_a, sem_b, sem_c):
    c = lax.axis_index("c")
    s = lax.axis_index("s")
    wid = c * NSUB + s
    stripe = n_pad // NSUB
    epw = nchunk * CH * BE
    imax = jnp.int32(n_pad - 1)

    zeros = jnp.zeros((16,), jnp.float32)

    def _zrow(r, carry):
        for cc in range(8):
            a_v[r, pl.ds(cc * 16, 16)] = zeros
        return carry

    lax.fori_loop(0, BE, _zrow, 0)
    _zero_fill(a_v, acc_s, s, stripe)
    plsc.subcore_barrier()

    def _mkclamp(idx_v):
        def _clamp(i, carry):
            r = i >> 3
            cc = (i & 7) * 16
            v = idx_v[r, pl.ds(cc, 16)]
            idx_v[r, pl.ds(cc, 16)] = jnp.minimum(jnp.maximum(v, 0), imax)
            return carry
        return _clamp

    def _chunk(t, carry):
        pltpu.sync_copy(src_hbm.at[wid, t], si_v)
        pltpu.sync_copy(dst_hbm.at[wid, t], di_v)
        lax.fori_loop(0, CH * 8, _mkclamp(si_v), 0)
        lax.fori_loop(0, CH * 8, _mkclamp(di_v), 0)

        def _addpass(base):
            def f(i, inner2):
                r = base + i * 4
                for rr in range(4):
                    for cc in range(8):
                        sl = pl.ds(cc * 16, 16)
                        a_v[r + rr, sl] = a_v[r + rr, sl] + b_v[r + rr, sl]
                return inner2
            lax.fori_loop(0, BE // 8, f, 0)

        def _silupass(base):
            def f(i, inner2):
                r = base + i * 4
                for rr in range(4):
                    for cc in range(8):
                        sl = pl.ds(cc * 16, 16)
                        tt = a_v[r + rr, sl] + b_v[r + rr, sl]
                        b_v[r + rr, sl] = tt / (1.0 + jnp.exp(-tt))
                return inner2
            lax.fori_loop(0, BE // 8, f, 0)

        H = BE // 2

        pltpu.async_copy(ps_hbm.at[si_v.at[0, pl.ds(0, H)]],
                         a_v.at[pl.ds(0, H)], sem_a)
        pltpu.async_copy(ps_hbm.at[si_v.at[0, pl.ds(H, H)]],
                         a_v.at[pl.ds(H, H)], sem_a)

        def _block(j, inner):
            rowb = wid * epw_real + (t * CH + j) * BE
            cpe0 = pltpu.async_copy(e_hbm.at[pl.ds(rowb, H)],
                                    b_v.at[pl.ds(0, H)], sem_b)
            cpe1 = pltpu.async_copy(e_hbm.at[pl.ds(rowb + H, H)],
                                    b_v.at[pl.ds(H, H)], sem_b)
            pltpu.make_async_copy(ps_hbm.at[si_v.at[j, pl.ds(0, H)]],
                                  a_v.at[pl.ds(0, H)], sem_a).wait()
            cpe0.wait()
            _addpass(0)
            cpd0 = pltpu.async_copy(pd_hbm.at[di_v.at[j, pl.ds(0, H)]],
                                    b_v.at[pl.ds(0, H)], sem_c)
            pltpu.make_async_copy(ps_hbm.at[si_v.at[j, pl.ds(H, H)]],
                                  a_v.at[pl.ds(H, H)], sem_a).wait()
            cpe1.wait()
            _addpass(H)
            cpd1 = pltpu.async_copy(pd_hbm.at[di_v.at[j, pl.ds(H, H)]],
                                    b_v.at[pl.ds(H, H)], sem_c)
            cpd0.wait()
            _silupass(0)
            cpd1.wait()
            _silupass(H)

            @pl.when(j < CH - 1)
            def _prefetch():
                pltpu.async_copy(ps_hbm.at[si_v.at[j + 1, pl.ds(0, H)]],
                                 a_v.at[pl.ds(0, H)], sem_a)
                pltpu.async_copy(ps_hbm.at[si_v.at[j + 1, pl.ds(H, H)]],
                                 a_v.at[pl.ds(H, H)], sem_a)

            pltpu.sync_copy(b_v, acc_s.at[di_v.at[j]], add=True)
            return inner

        lax.fori_loop(0, CH, _block, 0)
        return carry

    lax.fori_loop(0, nchunk, _chunk, 0)
    plsc.subcore_barrier()

    base = s * stripe
    pltpu.sync_copy(acc_s.at[pl.ds(base, stripe)],
                    outs_hbm.at[c, pl.ds(base, stripe)])


def _sc_count_body(n_pad, nchunk, dst_hbm, outc_hbm,
                   idx_v, one_v, acc_c):
    c = lax.axis_index("c")
    s = lax.axis_index("s")
    wid = c * NSUB + s
    stripe = n_pad // NSUB
    imax = jnp.int32(n_pad - 1)

    zeros = jnp.zeros((16,), jnp.float32)

    def _zrow(r, carry):
        for cc in range(8):
            one_v[r, pl.ds(cc * 16, 16)] = zeros
        return carry

    lax.fori_loop(0, BE, _zrow, 0)
    _zero_fill(one_v, acc_c, s, stripe)

    ones = jnp.ones((16,), jnp.float32)

    def _orow(r, carry):
        one_v[r, pl.ds(0, 16)] = ones
        return carry

    lax.fori_loop(0, BE, _orow, 0)
    plsc.subcore_barrier()

    def _clamp(i, carry):
        r = i >> 3
        cc = (i & 7) * 16
        v = idx_v[r, pl.ds(cc, 16)]
        idx_v[r, pl.ds(cc, 16)] = jnp.minimum(jnp.maximum(v, 0), imax)
        return carry

    def _chunk(t, carry):
        pltpu.sync_copy(dst_hbm.at[wid, t], idx_v)
        lax.fori_loop(0, CH * 8, _clamp, 0)

        def _block(j, inner):
            pltpu.sync_copy(one_v, acc_c.at[idx_v.at[j]], add=True)
            return inner

        lax.fori_loop(0, CH, _block, 0)
        return carry

    lax.fori_loop(0, nchunk, _chunk, 0)
    plsc.subcore_barrier()

    base = s * stripe
    pltpu.sync_copy(acc_c.at[pl.ds(base, stripe)],
                    outc_hbm.at[c, pl.ds(base, stripe)])


def _post_body(accs_ref, accc_ref, hd_ref, w2t_ref, b2_ref,
               wu1a_ref, wu1b_ref, bu1_ref, wu2t_ref, bu2_ref,
               g_ref, bt_ref, out_ref):
    seg = accs_ref[0] + accs_ref[1]
    cnt = (accc_ref[0] + accc_ref[1])[:, :1]
    agg_sum = jnp.dot(seg, w2t_ref[...],
                      preferred_element_type=jnp.float32) + cnt * b2_ref[...]
    agg = agg_sum / jnp.maximum(cnt, 1.0)
    hd = hd_ref[...]
    pre = (jnp.dot(hd, wu1a_ref[...], preferred_element_type=jnp.float32)
           + jnp.dot(agg, wu1b_ref[...], preferred_element_type=jnp.float32)
           + bu1_ref[...])
    dmid = _silu(pre)
    delta = jnp.dot(dmid, wu2t_ref[...],
                    preferred_element_type=jnp.float32) + bu2_ref[...]
    y = hd + delta
    mean = jnp.mean(y, axis=1, keepdims=True)
    d = y - mean
    var = jnp.mean(d * d, axis=1, keepdims=True)
    out_ref[...] = d * lax.rsqrt(var + 1e-5) * g_ref[...] + bt_ref[...]


def kernel(h_src, h_dst, edge_index, edge_attr, n_dst,
           W_msg1, b_msg1, W_msg2, b_msg2,
           W_upd1, b_upd1, W_upd2, b_upd2, gamma, beta):
    n_nodes = h_dst.shape[0]
    n_edges = edge_attr.shape[0]
    ef = edge_attr.shape[1]
    epw = n_edges // NW
    assert epw * NW == n_edges

    cbe = CH * BE
    epw_pad = -(-epw // cbe) * cbe
    nchunk = epw_pad // cbe
    stripe = -(-(-(-n_nodes // NSUB)) // 8) * 8
    n_pad = stripe * NSUB

    w1t = W_msg1.T
    w1a = w1t[:HID]
    w1b = w1t[HID:2 * HID]
    w1c = w1t[2 * HID:]

    f32 = jnp.float32
    hs_p = jnp.pad(h_src, ((0, n_pad - n_nodes), (0, 0)))
    hd_p = jnp.pad(h_dst, ((0, n_pad - n_nodes), (0, 0)))

    row_n = n_pad // 8
    p_src, p_dst = pl.pallas_call(
        _pre_node_body,
        grid=(8,),
        in_specs=[
            pl.BlockSpec((row_n, HID), lambda i: (i, 0)),
            pl.BlockSpec((row_n, HID), lambda i: (i, 0)),
            pl.BlockSpec((HID, HID), lambda i: (0, 0)),
            pl.BlockSpec((HID, HID), lambda i: (0, 0)),
            pl.BlockSpec((1, HID), lambda i: (0, 0)),
        ],
        out_specs=[
            pl.BlockSpec((row_n, HID), lambda i: (i, 0)),
            pl.BlockSpec((row_n, HID), lambda i: (i, 0)),
        ],
        out_shape=[
            jax.ShapeDtypeStruct((n_pad, HID), f32),
            jax.ShapeDtypeStruct((n_pad, HID), f32),
        ],
    )(hs_p, hd_p, w1a, w1b, b_msg1.reshape(1, HID))

    ne_pad = NW * epw_pad
    ea_p = jnp.pad(edge_attr, ((0, ne_pad - n_edges), (0, 0)))
    idx_pad = jnp.int32(n_pad - 1)
    src_r = jnp.pad(edge_index[0].astype(jnp.int32).reshape(NW, epw),
                    ((0, 0), (0, epw_pad - epw)),
                    constant_values=idx_pad).reshape(NW, nchunk, CH, BE)
    dst_r = jnp.pad(edge_index[1].astype(jnp.int32).reshape(NW, epw),
                    ((0, 0), (0, epw_pad - epw)),
                    constant_values=idx_pad).reshape(NW, nchunk, CH, BE)

    row_e = 4096
    grid_e = ne_pad // row_e
    assert grid_e * row_e == ne_pad
    e_proj = pl.pallas_call(
        _pre_edge_body,
        grid=(grid_e,),
        in_specs=[
            pl.BlockSpec((row_e, ef), lambda i: (i, 0)),
            pl.BlockSpec((ef, HID), lambda i: (0, 0)),
        ],
        out_specs=pl.BlockSpec((row_e, HID), lambda i: (i, 0)),
        out_shape=jax.ShapeDtypeStruct((ne_pad, HID), f32),
    )(ea_p, w1c)

    sc_edge = functools.partial(
        pl.kernel,
        out_type=jax.ShapeDtypeStruct((NCORE, n_pad, HID), f32),
        mesh=plsc.VectorSubcoreMesh(core_axis_name="c", subcore_axis_name="s"),
        scratch_types=[
            pltpu.MemorySpace.VMEM((CH, BE), jnp.int32),
            pltpu.MemorySpace.VMEM((CH, BE), jnp.int32),
            pltpu.MemorySpace.VMEM((BE, HID), f32),
            pltpu.MemorySpace.VMEM((BE, HID), f32),
            pltpu.MemorySpace.VMEM_SHARED((n_pad, HID), f32),
            pltpu.SemaphoreType.DMA,
            pltpu.SemaphoreType.DMA,
            pltpu.SemaphoreType.DMA,
        ],
    )(functools.partial(_sc_edge_body, n_pad, nchunk, epw))

    sc_count = functools.partial(
        pl.kernel,
        out_type=jax.ShapeDtypeStruct((NCORE, n_pad, HID), f32),
        mesh=plsc.VectorSubcoreMesh(core_axis_name="c", subcore_axis_name="s"),
        scratch_types=[
            pltpu.MemorySpace.VMEM((CH, BE), jnp.int32),
            pltpu.MemorySpace.VMEM((BE, HID), f32),
            pltpu.MemorySpace.VMEM_SHARED((n_pad, HID), f32),
        ],
    )(functools.partial(_sc_count_body, n_pad, nchunk))

    acc_c = sc_count(dst_r)

    acc_s = sc_edge(p_src, p_dst, e_proj, src_r, dst_r)

    wu1t = W_upd1.T
    row_p = n_nodes // 10
    out = pl.pallas_call(
        _post_body,
        grid=(10,),
        in_specs=[
            pl.BlockSpec((NCORE, row_p, HID), lambda i: (0, i, 0)),
            pl.BlockSpec((NCORE, row_p, HID), lambda i: (0, i, 0)),
            pl.BlockSpec((row_p, HID), lambda i: (i, 0)),
            pl.BlockSpec((HID, HID), lambda i: (0, 0)),
            pl.BlockSpec((1, HID), lambda i: (0, 0)),
            pl.BlockSpec((HID, HID), lambda i: (0, 0)),
            pl.BlockSpec((HID, HID), lambda i: (0, 0)),
            pl.BlockSpec((1, HID), lambda i: (0, 0)),
            pl.BlockSpec((HID, HID), lambda i: (0, 0)),
            pl.BlockSpec((1, HID), lambda i: (0, 0)),
            pl.BlockSpec((1, HID), lambda i: (0, 0)),
            pl.BlockSpec((1, HID), lambda i: (0, 0)),
        ],
        out_specs=pl.BlockSpec((row_p, HID), lambda i: (i, 0)),
        out_shape=jax.ShapeDtypeStruct((n_nodes, HID), f32),
    )(acc_s, acc_c, h_dst, W_msg2.T, b_msg2.reshape(1, HID),
      wu1t[:HID], wu1t[HID:], b_upd1.reshape(1, HID),
      W_upd2.T, b_upd2.reshape(1, HID),
      gamma.reshape(1, HID), beta.reshape(1, HID))
    return out


# speedup vs baseline: 2.7513x; 1.0075x over previous
import functools

import jax
import jax.numpy as jnp
from jax import lax
from jax.experimental import pallas as pl
from jax.experimental.pallas import tpu as pltpu
from jax.experimental.pallas import tpu_sc as plsc

HID = 128
NCORE = 2
NSUB = 16
NW = NCORE * NSUB
BE = 128
CH = 8


def _silu(x):
    return x / (1.0 + jnp.exp(-x))


def _pre_node_body(hs_ref, hd_ref, w1a_ref, w1b_ref, b1_ref, ps_ref, pd_ref):
    ps_ref[...] = jnp.dot(hs_ref[...], w1a_ref[...],
                          preferred_element_type=jnp.float32)
    pd_ref[...] = jnp.dot(hd_ref[...], w1b_ref[...],
                          preferred_element_type=jnp.float32) + b1_ref[...]


def _pre_edge_body(ea_ref, w1c_ref, e_ref):
    e_ref[...] = jnp.dot(ea_ref[...], w1c_ref[...],
                         preferred_element_type=jnp.float32)


def _zero_fill(a_v, ref, s, stripe):
    nrep = stripe // BE
    rem = stripe % BE
    for t in range(nrep):
        pltpu.sync_copy(a_v, ref.at[pl.ds(s * stripe + t * BE, BE)])
    if rem:
        pltpu.sync_copy(a_v.at[pl.ds(0, rem)],
                        ref.at[pl.ds(s * stripe + nrep * BE, rem)])


def _sc_edge_body(n_pad, nchunk, epw_real, ps_hbm, pd_hbm, e_hbm, src_hbm, dst_hbm,
                  outs_hbm,
                  si_v, di_v, a_v, b_v,
                  acc_s, sem_a, sem_b, sem_c):
    c = lax.axis_index("c")
    s = lax.axis_index("s")
    wid = c * NSUB + s
    stripe = n_pad // NSUB
    epw = nchunk * CH * BE
    imax = jnp.int32(n_pad - 1)

    zeros = jnp.zeros((16,), jnp.float32)

    def _zrow(r, carry):
        for cc in range(8):
            a_v[r, pl.ds(cc * 16, 16)] = zeros
        return carry

    lax.fori_loop(0, BE, _zrow, 0)
    _zero_fill(a_v, acc_s, s, stripe)
    plsc.subcore_barrier()

    def _chunk(t, carry):
        pltpu.sync_copy(src_hbm.at[wid, t], si_v)
        pltpu.sync_copy(dst_hbm.at[wid, t], di_v)

        def _addpass(base):
            def f(i, inner2):
                r = base + i * 4
                for rr in range(4):
                    for cc in range(8):
                        sl = pl.ds(cc * 16, 16)
                        a_v[r + rr, sl] = a_v[r + rr, sl] + b_v[r + rr, sl]
                return inner2
            lax.fori_loop(0, BE // 8, f, 0)

        def _silupass(base):
            def f(i, inner2):
                r = base + i * 4
                for rr in range(4):
                    for cc in range(8):
                        sl = pl.ds(cc * 16, 16)
                        tt = a_v[r + rr, sl] + b_v[r + rr, sl]
                        b_v[r + rr, sl] = tt / (1.0 + jnp.exp(-tt))
                return inner2
            lax.fori_loop(0, BE // 8, f, 0)

        H = BE // 2

        pltpu.async_copy(ps_hbm.at[si_v.at[0, pl.ds(0, H)]],
                         a_v.at[pl.ds(0, H)], sem_a)
        pltpu.async_copy(ps_hbm.at[si_v.at[0, pl.ds(H, H)]],
                         a_v.at[pl.ds(H, H)], sem_a)

        def _block(j, inner):
            rowb = wid * epw_real + (t * CH + j) * BE
            cpe0 = pltpu.async_copy(e_hbm.at[pl.ds(rowb, H)],
                                    b_v.at[pl.ds(0, H)], sem_b)
            cpe1 = pltpu.async_copy(e_hbm.at[pl.ds(rowb + H, H)],
                                    b_v.at[pl.ds(H, H)], sem_b)
            pltpu.make_async_copy(ps_hbm.at[si_v.at[j, pl.ds(0, H)]],
                                  a_v.at[pl.ds(0, H)], sem_a).wait()
            cpe0.wait()
            _addpass(0)
            cpd0 = pltpu.async_copy(pd_hbm.at[di_v.at[j, pl.ds(0, H)]],
                                    b_v.at[pl.ds(0, H)], sem_c)
            pltpu.make_async_copy(ps_hbm.at[si_v.at[j, pl.ds(H, H)]],
                                  a_v.at[pl.ds(H, H)], sem_a).wait()
            cpe1.wait()
            _addpass(H)
            cpd1 = pltpu.async_copy(pd_hbm.at[di_v.at[j, pl.ds(H, H)]],
                                    b_v.at[pl.ds(H, H)], sem_c)
            cpd0.wait()
            _silupass(0)
            cpd1.wait()
            _silupass(H)

            @pl.when(j < CH - 1)
            def _prefetch():
                pltpu.async_copy(ps_hbm.at[si_v.at[j + 1, pl.ds(0, H)]],
                                 a_v.at[pl.ds(0, H)], sem_a)
                pltpu.async_copy(ps_hbm.at[si_v.at[j + 1, pl.ds(H, H)]],
                                 a_v.at[pl.ds(H, H)], sem_a)

            pltpu.sync_copy(b_v, acc_s.at[di_v.at[j]], add=True)
            return inner

        lax.fori_loop(0, CH, _block, 0)
        return carry

    lax.fori_loop(0, nchunk, _chunk, 0)
    plsc.subcore_barrier()

    base = s * stripe
    pltpu.sync_copy(acc_s.at[pl.ds(base, stripe)],
                    outs_hbm.at[c, pl.ds(base, stripe)])


def _sc_count_body(n_pad, nchunk, dst_hbm, outc_hbm,
                   idx_v, one_v, acc_c):
    c = lax.axis_index("c")
    s = lax.axis_index("s")
    wid = c * NSUB + s
    stripe = n_pad // NSUB
    imax = jnp.int32(n_pad - 1)

    zeros = jnp.zeros((16,), jnp.float32)

    def _zrow(r, carry):
        for cc in range(8):
            one_v[r, pl.ds(cc * 16, 16)] = zeros
        return carry

    lax.fori_loop(0, BE, _zrow, 0)
    _zero_fill(one_v, acc_c, s, stripe)

    ones = jnp.ones((16,), jnp.float32)

    def _orow(r, carry):
        one_v[r, pl.ds(0, 16)] = ones
        return carry

    lax.fori_loop(0, BE, _orow, 0)
    plsc.subcore_barrier()

    def _chunk(t, carry):
        pltpu.sync_copy(dst_hbm.at[wid, t], idx_v)

        def _block(j, inner):
            pltpu.sync_copy(one_v, acc_c.at[idx_v.at[j]], add=True)
            return inner

        lax.fori_loop(0, CH, _block, 0)
        return carry

    lax.fori_loop(0, nchunk, _chunk, 0)
    plsc.subcore_barrier()

    base = s * stripe
    pltpu.sync_copy(acc_c.at[pl.ds(base, stripe)],
                    outc_hbm.at[c, pl.ds(base, stripe)])


def _post_body(accs_ref, accc_ref, hd_ref, w2t_ref, b2_ref,
               wu1a_ref, wu1b_ref, bu1_ref, wu2t_ref, bu2_ref,
               g_ref, bt_ref, out_ref):
    seg = accs_ref[0] + accs_ref[1]
    cnt = (accc_ref[0] + accc_ref[1])[:, :1]
    agg_sum = jnp.dot(seg, w2t_ref[...],
                      preferred_element_type=jnp.float32) + cnt * b2_ref[...]
    agg = agg_sum / jnp.maximum(cnt, 1.0)
    hd = hd_ref[...]
    pre = (jnp.dot(hd, wu1a_ref[...], preferred_element_type=jnp.float32)
           + jnp.dot(agg, wu1b_ref[...], preferred_element_type=jnp.float32)
           + bu1_ref[...])
    dmid = _silu(pre)
    delta = jnp.dot(dmid, wu2t_ref[...],
                    preferred_element_type=jnp.float32) + bu2_ref[...]
    y = hd + delta
    mean = jnp.mean(y, axis=1, keepdims=True)
    d = y - mean
    var = jnp.mean(d * d, axis=1, keepdims=True)
    out_ref[...] = d * lax.rsqrt(var + 1e-5) * g_ref[...] + bt_ref[...]


def kernel(h_src, h_dst, edge_index, edge_attr, n_dst,
           W_msg1, b_msg1, W_msg2, b_msg2,
           W_upd1, b_upd1, W_upd2, b_upd2, gamma, beta):
    n_nodes = h_dst.shape[0]
    n_edges = edge_attr.shape[0]
    ef = edge_attr.shape[1]
    epw = n_edges // NW
    assert epw * NW == n_edges

    cbe = CH * BE
    epw_pad = -(-epw // cbe) * cbe
    nchunk = epw_pad // cbe
    stripe = -(-(-(-n_nodes // NSUB)) // 8) * 8
    n_pad = stripe * NSUB

    w1t = W_msg1.T
    w1a = w1t[:HID]
    w1b = w1t[HID:2 * HID]
    w1c = w1t[2 * HID:]

    f32 = jnp.float32
    hs_p = jnp.pad(h_src, ((0, n_pad - n_nodes), (0, 0)))
    hd_p = jnp.pad(h_dst, ((0, n_pad - n_nodes), (0, 0)))

    row_n = n_pad // 8
    p_src, p_dst = pl.pallas_call(
        _pre_node_body,
        grid=(8,),
        in_specs=[
            pl.BlockSpec((row_n, HID), lambda i: (i, 0)),
            pl.BlockSpec((row_n, HID), lambda i: (i, 0)),
            pl.BlockSpec((HID, HID), lambda i: (0, 0)),
            pl.BlockSpec((HID, HID), lambda i: (0, 0)),
            pl.BlockSpec((1, HID), lambda i: (0, 0)),
        ],
        out_specs=[
            pl.BlockSpec((row_n, HID), lambda i: (i, 0)),
            pl.BlockSpec((row_n, HID), lambda i: (i, 0)),
        ],
        out_shape=[
            jax.ShapeDtypeStruct((n_pad, HID), f32),
            jax.ShapeDtypeStruct((n_pad, HID), f32),
        ],
    )(hs_p, hd_p, w1a, w1b, b_msg1.reshape(1, HID))

    ne_pad = NW * epw_pad
    ea_p = jnp.pad(edge_attr, ((0, ne_pad - n_edges), (0, 0)))
    idx_pad = jnp.int32(n_pad - 1)
    src_r = jnp.pad(edge_index[0].astype(jnp.int32).reshape(NW, epw),
                    ((0, 0), (0, epw_pad - epw)),
                    constant_values=idx_pad).reshape(NW, nchunk, CH, BE)
    dst_r = jnp.pad(edge_index[1].astype(jnp.int32).reshape(NW, epw),
                    ((0, 0), (0, epw_pad - epw)),
                    constant_values=idx_pad).reshape(NW, nchunk, CH, BE)

    row_e = 4096
    grid_e = ne_pad // row_e
    assert grid_e * row_e == ne_pad
    e_proj = pl.pallas_call(
        _pre_edge_body,
        grid=(grid_e,),
        in_specs=[
            pl.BlockSpec((row_e, ef), lambda i: (i, 0)),
            pl.BlockSpec((ef, HID), lambda i: (0, 0)),
        ],
        out_specs=pl.BlockSpec((row_e, HID), lambda i: (i, 0)),
        out_shape=jax.ShapeDtypeStruct((ne_pad, HID), f32),
    )(ea_p, w1c)

    sc_edge = functools.partial(
        pl.kernel,
        out_type=jax.ShapeDtypeStruct((NCORE, n_pad, HID), f32),
        mesh=plsc.VectorSubcoreMesh(core_axis_name="c", subcore_axis_name="s"),
        scratch_types=[
            pltpu.MemorySpace.VMEM((CH, BE), jnp.int32),
            pltpu.MemorySpace.VMEM((CH, BE), jnp.int32),
            pltpu.MemorySpace.VMEM((BE, HID), f32),
            pltpu.MemorySpace.VMEM((BE, HID), f32),
            pltpu.MemorySpace.VMEM_SHARED((n_pad, HID), f32),
            pltpu.SemaphoreType.DMA,
            pltpu.SemaphoreType.DMA,
            pltpu.SemaphoreType.DMA,
        ],
    )(functools.partial(_sc_edge_body, n_pad, nchunk, epw))

    sc_count = functools.partial(
        pl.kernel,
        out_type=jax.ShapeDtypeStruct((NCORE, n_pad, HID), f32),
        mesh=plsc.VectorSubcoreMesh(core_axis_name="c", subcore_axis_name="s"),
        scratch_types=[
            pltpu.MemorySpace.VMEM((CH, BE), jnp.int32),
            pltpu.MemorySpace.VMEM((BE, HID), f32),
            pltpu.MemorySpace.VMEM_SHARED((n_pad, HID), f32),
        ],
    )(functools.partial(_sc_count_body, n_pad, nchunk))

    acc_c = sc_count(dst_r)

    acc_s = sc_edge(p_src, p_dst, e_proj, src_r, dst_r)

    wu1t = W_upd1.T
    row_p = n_nodes // 10
    out = pl.pallas_call(
        _post_body,
        grid=(10,),
        in_specs=[
            pl.BlockSpec((NCORE, row_p, HID), lambda i: (0, i, 0)),
            pl.BlockSpec((NCORE, row_p, HID), lambda i: (0, i, 0)),
            pl.BlockSpec((row_p, HID), lambda i: (i, 0)),
            pl.BlockSpec((HID, HID), lambda i: (0, 0)),
            pl.BlockSpec((1, HID), lambda i: (0, 0)),
            pl.BlockSpec((HID, HID), lambda i: (0, 0)),
            pl.BlockSpec((HID, HID), lambda i: (0, 0)),
            pl.BlockSpec((1, HID), lambda i: (0, 0)),
            pl.BlockSpec((HID, HID), lambda i: (0, 0)),
            pl.BlockSpec((1, HID), lambda i: (0, 0)),
            pl.BlockSpec((1, HID), lambda i: (0, 0)),
            pl.BlockSpec((1, HID), lambda i: (0, 0)),
        ],
        out_specs=pl.BlockSpec((row_p, HID), lambda i: (i, 0)),
        out_shape=jax.ShapeDtypeStruct((n_nodes, HID), f32),
    )(acc_s, acc_c, h_dst, W_msg2.T, b_msg2.reshape(1, HID),
      wu1t[:HID], wu1t[HID:], b_upd1.reshape(1, HID),
      W_upd2.T, b_upd2.reshape(1, HID),
      gamma.reshape(1, HID), beta.reshape(1, HID))
    return out
